# Initial kernel scaffold; baseline (speedup 1.0000x reference)
import jax
import jax.numpy as jnp
from jax.experimental import pallas as pl


def kernel(h, edge_index, edge_attr, W1, b1, W2, b2, W3, b3, W4, b4):
    raise NotImplementedError("write your pallas kernel here")


# speedup vs baseline: 2.3594x; 2.3594x over previous
import functools

import jax
import jax.numpy as jnp
from jax import lax
from jax.experimental import pallas as pl
from jax.experimental.pallas import tpu as pltpu
from jax.experimental.pallas import tpu_sc as plsc

N_NODES = 10000
N_EDGES = 320000
D_FEAT = 128
D_EDGE = 16
HIDDEN = 256

NC = 2
NS = 16
NW = NC * NS


E_PER_W = N_EDGES // NW
GCHUNK = 80
GITERS = E_PER_W // GCHUNK

_sc_mesh = plsc.VectorSubcoreMesh(core_axis_name="c", subcore_axis_name="s")


@functools.partial(
    pl.kernel,
    mesh=_sc_mesh,
    out_type=[
        jax.ShapeDtypeStruct((N_EDGES, D_FEAT), jnp.float32),
        jax.ShapeDtypeStruct((N_EDGES, D_FEAT), jnp.float32),
    ],
    scratch_types=[
        pltpu.VMEM((E_PER_W,), jnp.int32),
        pltpu.VMEM((E_PER_W,), jnp.int32),
        pltpu.VMEM((GCHUNK, D_FEAT), jnp.float32),
        pltpu.VMEM((GCHUNK, D_FEAT), jnp.float32),
        pltpu.SemaphoreType.DMA,
        pltpu.SemaphoreType.DMA,
    ],
)
def _sc_gather(h_hbm, row_hbm, col_hbm, src_hbm, tgt_hbm,
               rowi, coli, bufa, bufb, sema, semb):
    c = lax.axis_index("c")
    s = lax.axis_index("s")
    wid = s * NC + c
    base = wid * E_PER_W
    pltpu.sync_copy(row_hbm.at[pl.ds(base, E_PER_W)], rowi)
    pltpu.sync_copy(col_hbm.at[pl.ds(base, E_PER_W)], coli)

    def body(j, carry):
        off = j * GCHUNK
        cpa = pltpu.async_copy(h_hbm.at[rowi.at[pl.ds(off, GCHUNK)]], bufa, sema)
        cpb = pltpu.async_copy(h_hbm.at[coli.at[pl.ds(off, GCHUNK)]], bufb, semb)
        cpa.wait()
        pltpu.sync_copy(bufa, src_hbm.at[pl.ds(base + off, GCHUNK)])
        cpb.wait()
        pltpu.sync_copy(bufb, tgt_hbm.at[pl.ds(base + off, GCHUNK)])
        return carry

    lax.fori_loop(0, GITERS, body, 0)


E_PER_T = N_EDGES // NS
SCHUNK = 80
SITERS = E_PER_T // SCHUNK
HALF = HIDDEN // NC
NPASS = 2
NHALF = N_NODES // NPASS
ACC_ROWS = NHALF + 8
ZCHUNK = 16
OCHUNK = 40


@functools.partial(
    pl.kernel,
    mesh=_sc_mesh,
    out_type=jax.ShapeDtypeStruct((N_NODES, HIDDEN), jnp.float32),
    scratch_types=[
        pltpu.VMEM((SITERS, SCHUNK), jnp.int32),
        pltpu.VMEM((1, SCHUNK), jnp.int32),
        pltpu.VMEM((SCHUNK, HALF), jnp.float32),
        pltpu.VMEM((ZCHUNK, HALF), jnp.float32),
        pltpu.VMEM((OCHUNK, HALF), jnp.float32),
        pltpu.VMEM_SHARED((ACC_ROWS, HALF), jnp.float32),
    ],
)
def _sc_scatter(mij_hbm, row3_hbm, agg_hbm, idxv, lidx, bufv, zbuf, obuf, acc):
    c = lax.axis_index("c")
    s = lax.axis_index("s")

    zero = jnp.zeros((16,), jnp.float32)

    def zrow(i, carry):
        for q in range(HALF // 16):
            zbuf[i, pl.ds(q * 16, 16)] = zero
        return carry

    lax.fori_loop(0, ZCHUNK, zrow, 0)

    pltpu.sync_copy(row3_hbm.at[s], idxv)

    ziters = jnp.where(s == NS - 1, 13, 20)
    oiters = jnp.where(s == NS - 1, 5, 8)
    zstart = s * 320

    for p in range(NPASS):
        nbase = p * NHALF

        def zcp(t, carry):
            pltpu.sync_copy(zbuf, acc.at[pl.ds(zstart + t * ZCHUNK, ZCHUNK)])
            return carry

        lax.fori_loop(0, ziters, zcp, 0)
        plsc.subcore_barrier()

        def ebody(j, carry):
            ebase = s * E_PER_T + j * SCHUNK
            pltpu.sync_copy(
                mij_hbm.at[pl.ds(ebase, SCHUNK), pl.ds(c * HALF, HALF)], bufv)
            for k in range(SCHUNK // 16):
                v = idxv[j, pl.ds(k * 16, 16)] - nbase
                ok = (v >= 0) & (v < NHALF)
                lidx[0, pl.ds(k * 16, 16)] = jnp.where(ok, v, NHALF)
            pltpu.sync_copy(bufv, acc.at[lidx.at[0]], add=True)
            return carry

        lax.fori_loop(0, SITERS, ebody, 0)
        plsc.subcore_barrier()

        def obody(t, carry):
            rbase = zstart + t * OCHUNK
            pltpu.sync_copy(acc.at[pl.ds(rbase, OCHUNK)], obuf)
            pltpu.sync_copy(
                obuf,
                agg_hbm.at[pl.ds(nbase + rbase, OCHUNK), pl.ds(c * HALF, HALF)])
            return carry

        lax.fori_loop(0, oiters, obody, 0)


BE = 3200


def _edge_mlp_body(src_ref, tgt_ref, ea_ref, w1a, w1b, w1c, b1r, w2r, b2r,
                   out_ref):
    x = (jnp.dot(src_ref[...], w1a[...], preferred_element_type=jnp.float32)
         + jnp.dot(tgt_ref[...], w1b[...], preferred_element_type=jnp.float32)
         + jnp.dot(ea_ref[...], w1c[...], preferred_element_type=jnp.float32)
         + b1r[...])
    x = x * jax.nn.sigmoid(x)
    y = jnp.dot(x, w2r[...], preferred_element_type=jnp.float32) + b2r[...]
    out_ref[...] = y * jax.nn.sigmoid(y)


def _edge_mlp(src, tgt, ea, w1a, w1b, w1c, b1, w2, b2):
    grid = (N_EDGES // BE,)
    full = lambda shape: pl.BlockSpec(shape, lambda i: (0, 0))
    return pl.pallas_call(
        _edge_mlp_body,
        grid=grid,
        in_specs=[
            pl.BlockSpec((BE, D_FEAT), lambda i: (i, 0)),
            pl.BlockSpec((BE, D_FEAT), lambda i: (i, 0)),
            pl.BlockSpec((BE, D_EDGE), lambda i: (i, 0)),
            full((D_FEAT, HIDDEN)),
            full((D_FEAT, HIDDEN)),
            full((D_EDGE, HIDDEN)),
            full((1, HIDDEN)),
            full((HIDDEN, HIDDEN)),
            full((1, HIDDEN)),
        ],
        out_specs=pl.BlockSpec((BE, HIDDEN), lambda i: (i, 0)),
        out_shape=jax.ShapeDtypeStruct((N_EDGES, HIDDEN), jnp.float32),
    )(src, tgt, ea, w1a, w1b, w1c, b1, w2, b2)


BN = 2000


def _node_mlp_body(h_ref, agg_ref, w3a, w3b, b3r, w4r, b4r, out_ref):
    x = (jnp.dot(h_ref[...], w3a[...], preferred_element_type=jnp.float32)
         + jnp.dot(agg_ref[...], w3b[...], preferred_element_type=jnp.float32)
         + b3r[...])
    x = x * jax.nn.sigmoid(x)
    out_ref[...] = (h_ref[...]
                    + jnp.dot(x, w4r[...], preferred_element_type=jnp.float32)
                    + b4r[...])


def _node_mlp(h, agg, w3a, w3b, b3, w4, b4):
    grid = (N_NODES // BN,)
    full = lambda shape: pl.BlockSpec(shape, lambda i: (0, 0))
    return pl.pallas_call(
        _node_mlp_body,
        grid=grid,
        in_specs=[
            pl.BlockSpec((BN, D_FEAT), lambda i: (i, 0)),
            pl.BlockSpec((BN, HIDDEN), lambda i: (i, 0)),
            full((D_FEAT, HIDDEN)),
            full((HIDDEN, HIDDEN)),
            full((1, HIDDEN)),
            full((HIDDEN, D_FEAT)),
            full((1, D_FEAT)),
        ],
        out_specs=pl.BlockSpec((BN, D_FEAT), lambda i: (i, 0)),
        out_shape=jax.ShapeDtypeStruct((N_NODES, D_FEAT), jnp.float32),
    )(h, agg, w3a, w3b, b3, w4, b4)


def kernel(h, edge_index, edge_attr, W1, b1, W2, b2, W3, b3, W4, b4):
    row = edge_index[0].astype(jnp.int32)
    col = edge_index[1].astype(jnp.int32)

    src, tgt = _sc_gather(h, row, col)

    mij = _edge_mlp(src, tgt, edge_attr,
                    W1[:D_FEAT], W1[D_FEAT:2 * D_FEAT], W1[2 * D_FEAT:],
                    b1.reshape(1, -1), W2, b2.reshape(1, -1))

    row3 = row.reshape(NS, SITERS, SCHUNK)
    agg = _sc_scatter(mij, row3)

    h_new = _node_mlp(h, agg, W3[:D_FEAT], W3[D_FEAT:],
                      b3.reshape(1, -1), W4, b4.reshape(1, -1))
    return (h_new, mij)


# speedup vs baseline: 2.7379x; 1.1605x over previous
import functools

import jax
import jax.numpy as jnp
from jax import lax
from jax.experimental import pallas as pl
from jax.experimental.pallas import tpu as pltpu
from jax.experimental.pallas import tpu_sc as plsc

N_NODES = 10000
N_EDGES = 320000
D_FEAT = 128
D_EDGE = 16
HIDDEN = 256

NC = 2
NS = 16
NW = NC * NS


E_PER_W = N_EDGES // NW
GCHUNK = 80
GITERS = E_PER_W // GCHUNK

_sc_mesh = plsc.VectorSubcoreMesh(core_axis_name="c", subcore_axis_name="s")


@functools.partial(
    pl.kernel,
    mesh=_sc_mesh,
    out_type=[
        jax.ShapeDtypeStruct((N_EDGES, D_FEAT), jnp.float32),
        jax.ShapeDtypeStruct((N_EDGES, D_FEAT), jnp.float32),
    ],
    scratch_types=[
        pltpu.VMEM((E_PER_W,), jnp.int32),
        pltpu.VMEM((E_PER_W,), jnp.int32),
        pltpu.VMEM((GCHUNK, D_FEAT), jnp.float32),
        pltpu.VMEM((GCHUNK, D_FEAT), jnp.float32),
        pltpu.VMEM((GCHUNK, D_FEAT), jnp.float32),
        pltpu.VMEM((GCHUNK, D_FEAT), jnp.float32),
        pltpu.SemaphoreType.DMA,
        pltpu.SemaphoreType.DMA,
        pltpu.SemaphoreType.DMA,
        pltpu.SemaphoreType.DMA,
    ],
)
def _sc_gather(h_hbm, row_hbm, col_hbm, src_hbm, tgt_hbm,
               rowi, coli, bufa0, bufb0, bufa1, bufb1, sa0, sb0, sa1, sb1):
    c = lax.axis_index("c")
    s = lax.axis_index("s")
    wid = s * NC + c
    base = wid * E_PER_W
    pltpu.sync_copy(row_hbm.at[pl.ds(base, E_PER_W)], rowi)
    pltpu.sync_copy(col_hbm.at[pl.ds(base, E_PER_W)], coli)

    def issue(off, ba, bb, sema, semb):
        pltpu.async_copy(h_hbm.at[rowi.at[pl.ds(off, GCHUNK)]], ba, sema)
        pltpu.async_copy(h_hbm.at[coli.at[pl.ds(off, GCHUNK)]], bb, semb)

    def drain(ba, bb, sema, semb):
        pltpu.make_async_copy(h_hbm.at[pl.ds(0, GCHUNK)], ba, sema).wait()
        pltpu.make_async_copy(h_hbm.at[pl.ds(0, GCHUNK)], bb, semb).wait()

    def write(off, ba, bb):
        pltpu.sync_copy(ba, src_hbm.at[pl.ds(base + off, GCHUNK)])
        pltpu.sync_copy(bb, tgt_hbm.at[pl.ds(base + off, GCHUNK)])

    issue(0, bufa0, bufb0, sa0, sb0)

    def body(t, carry):
        off0 = 2 * t * GCHUNK
        off1 = off0 + GCHUNK
        issue(off1, bufa1, bufb1, sa1, sb1)
        drain(bufa0, bufb0, sa0, sb0)
        write(off0, bufa0, bufb0)
        issue(off1 + GCHUNK, bufa0, bufb0, sa0, sb0)
        drain(bufa1, bufb1, sa1, sb1)
        write(off1, bufa1, bufb1)
        return carry

    lax.fori_loop(0, (GITERS - 1) // 2, body, 0)
    drain(bufa0, bufb0, sa0, sb0)
    write((GITERS - 1) * GCHUNK, bufa0, bufb0)


E_PER_T = N_EDGES // NS
SCHUNK = 80
SITERS = E_PER_T // SCHUNK
HALF = HIDDEN // NC
NPASS = 2
NHALF = N_NODES // NPASS
ACC_ROWS = NHALF + 8
ZCHUNK = 16
OCHUNK = 40


@functools.partial(
    pl.kernel,
    mesh=_sc_mesh,
    out_type=jax.ShapeDtypeStruct((N_NODES, HIDDEN), jnp.float32),
    scratch_types=[
        pltpu.VMEM((SITERS, SCHUNK), jnp.int32),
        pltpu.VMEM((1, SCHUNK), jnp.int32),
        pltpu.VMEM((1, SCHUNK), jnp.int32),
        pltpu.VMEM((SCHUNK, HALF), jnp.float32),
        pltpu.VMEM((SCHUNK, HALF), jnp.float32),
        pltpu.VMEM((ZCHUNK, HALF), jnp.float32),
        pltpu.VMEM((OCHUNK, HALF), jnp.float32),
        pltpu.VMEM_SHARED((ACC_ROWS, HALF), jnp.float32),
        pltpu.SemaphoreType.DMA,
        pltpu.SemaphoreType.DMA,
        pltpu.SemaphoreType.DMA,
        pltpu.SemaphoreType.DMA,
    ],
)
def _sc_scatter(mij_hbm, row3_hbm, agg_hbm, idxv, lidx0, lidx1,
                bufv0, bufv1, zbuf, obuf, acc, sr0, sr1, ss0, ss1):
    c = lax.axis_index("c")
    s = lax.axis_index("s")

    zero = jnp.zeros((16,), jnp.float32)

    def zrow(i, carry):
        for q in range(HALF // 16):
            zbuf[i, pl.ds(q * 16, 16)] = zero
        return carry

    lax.fori_loop(0, ZCHUNK, zrow, 0)

    pltpu.sync_copy(row3_hbm.at[s], idxv)

    ziters = jnp.where(s == NS - 1, 13, 20)
    oiters = jnp.where(s == NS - 1, 5, 8)
    zstart = s * 320

    for p in range(NPASS):
        nbase = p * NHALF

        def zcp(t, carry):
            pltpu.sync_copy(zbuf, acc.at[pl.ds(zstart + t * ZCHUNK, ZCHUNK)])
            return carry

        lax.fori_loop(0, ziters, zcp, 0)
        plsc.subcore_barrier()

        def rd_issue(j, buf, sem):
            ebase = s * E_PER_T + j * SCHUNK
            pltpu.async_copy(
                mij_hbm.at[pl.ds(ebase, SCHUNK), pl.ds(c * HALF, HALF)],
                buf, sem)

        def rd_wait(buf, sem):
            pltpu.make_async_copy(
                mij_hbm.at[pl.ds(0, SCHUNK), pl.ds(0, HALF)], buf, sem).wait()

        def remap(j, lidx):
            for k in range(SCHUNK // 16):
                v = idxv[j, pl.ds(k * 16, 16)] - nbase
                ok = (v >= 0) & (v < NHALF)
                lidx[0, pl.ds(k * 16, 16)] = jnp.where(ok, v, NHALF)

        def sc_wait(buf, sem):
            pltpu.make_async_copy(buf, acc.at[pl.ds(0, SCHUNK)], sem).wait()

        rd_issue(0, bufv0, sr0)

        def ebody(t, carry):
            j0 = 2 * t
            j1 = j0 + 1

            @pl.when(t > 0)
            def _():
                sc_wait(bufv1, ss1)

            rd_issue(j1, bufv1, sr1)
            rd_wait(bufv0, sr0)
            remap(j0, lidx0)
            pltpu.async_copy(bufv0, acc.at[lidx0.at[0]], ss0, add=True)
            rd_wait(bufv1, sr1)
            remap(j1, lidx1)
            pltpu.async_copy(bufv1, acc.at[lidx1.at[0]], ss1, add=True)
            sc_wait(bufv0, ss0)

            @pl.when(t < SITERS // 2 - 1)
            def _():
                rd_issue(j0 + 2, bufv0, sr0)

            return carry

        lax.fori_loop(0, SITERS // 2, ebody, 0)
        sc_wait(bufv1, ss1)
        plsc.subcore_barrier()

        def obody(t, carry):
            rbase = zstart + t * OCHUNK
            pltpu.sync_copy(acc.at[pl.ds(rbase, OCHUNK)], obuf)
            pltpu.sync_copy(
                obuf,
                agg_hbm.at[pl.ds(nbase + rbase, OCHUNK), pl.ds(c * HALF, HALF)])
            return carry

        lax.fori_loop(0, oiters, obody, 0)


BE = 3200


def _edge_mlp_body(src_ref, tgt_ref, ea_ref, w1a, w1b, w1c, b1r, w2r, b2r,
                   out_ref):
    x = (jnp.dot(src_ref[...], w1a[...], preferred_element_type=jnp.float32)
         + jnp.dot(tgt_ref[...], w1b[...], preferred_element_type=jnp.float32)
         + jnp.dot(ea_ref[...], w1c[...], preferred_element_type=jnp.float32)
         + b1r[...])
    x = x * jax.nn.sigmoid(x)
    y = jnp.dot(x, w2r[...], preferred_element_type=jnp.float32) + b2r[...]
    out_ref[...] = y * jax.nn.sigmoid(y)


def _edge_mlp(src, tgt, ea, w1a, w1b, w1c, b1, w2, b2):
    grid = (N_EDGES // BE,)
    full = lambda shape: pl.BlockSpec(shape, lambda i: (0, 0))
    return pl.pallas_call(
        _edge_mlp_body,
        grid=grid,
        in_specs=[
            pl.BlockSpec((BE, D_FEAT), lambda i: (i, 0)),
            pl.BlockSpec((BE, D_FEAT), lambda i: (i, 0)),
            pl.BlockSpec((BE, D_EDGE), lambda i: (i, 0)),
            full((D_FEAT, HIDDEN)),
            full((D_FEAT, HIDDEN)),
            full((D_EDGE, HIDDEN)),
            full((1, HIDDEN)),
            full((HIDDEN, HIDDEN)),
            full((1, HIDDEN)),
        ],
        out_specs=pl.BlockSpec((BE, HIDDEN), lambda i: (i, 0)),
        out_shape=jax.ShapeDtypeStruct((N_EDGES, HIDDEN), jnp.float32),
    )(src, tgt, ea, w1a, w1b, w1c, b1, w2, b2)


BN = 2000


def _node_mlp_body(h_ref, agg_ref, w3a, w3b, b3r, w4r, b4r, out_ref):
    x = (jnp.dot(h_ref[...], w3a[...], preferred_element_type=jnp.float32)
         + jnp.dot(agg_ref[...], w3b[...], preferred_element_type=jnp.float32)
         + b3r[...])
    x = x * jax.nn.sigmoid(x)
    out_ref[...] = (h_ref[...]
                    + jnp.dot(x, w4r[...], preferred_element_type=jnp.float32)
                    + b4r[...])


def _node_mlp(h, agg, w3a, w3b, b3, w4, b4):
    grid = (N_NODES // BN,)
    full = lambda shape: pl.BlockSpec(shape, lambda i: (0, 0))
    return pl.pallas_call(
        _node_mlp_body,
        grid=grid,
        in_specs=[
            pl.BlockSpec((BN, D_FEAT), lambda i: (i, 0)),
            pl.BlockSpec((BN, HIDDEN), lambda i: (i, 0)),
            full((D_FEAT, HIDDEN)),
            full((HIDDEN, HIDDEN)),
            full((1, HIDDEN)),
            full((HIDDEN, D_FEAT)),
            full((1, D_FEAT)),
        ],
        out_specs=pl.BlockSpec((BN, D_FEAT), lambda i: (i, 0)),
        out_shape=jax.ShapeDtypeStruct((N_NODES, D_FEAT), jnp.float32),
    )(h, agg, w3a, w3b, b3, w4, b4)


def kernel(h, edge_index, edge_attr, W1, b1, W2, b2, W3, b3, W4, b4):
    row = edge_index[0].astype(jnp.int32)
    col = edge_index[1].astype(jnp.int32)

    src, tgt = _sc_gather(h, row, col)

    mij = _edge_mlp(src, tgt, edge_attr,
                    W1[:D_FEAT], W1[D_FEAT:2 * D_FEAT], W1[2 * D_FEAT:],
                    b1.reshape(1, -1), W2, b2.reshape(1, -1))

    row3 = row.reshape(NS, SITERS, SCHUNK)
    agg = _sc_scatter(mij, row3)

    h_new = _node_mlp(h, agg, W3[:D_FEAT], W3[D_FEAT:],
                      b3.reshape(1, -1), W4, b4.reshape(1, -1))
    return (h_new, mij)


# speedup vs baseline: 2.9178x; 1.0657x over previous
import functools

import jax
import jax.numpy as jnp
from jax import lax
from jax.experimental import pallas as pl
from jax.experimental.pallas import tpu as pltpu
from jax.experimental.pallas import tpu_sc as plsc

N_NODES = 10000
N_EDGES = 320000
D_FEAT = 128
D_EDGE = 16
HIDDEN = 256

NC = 2
NS = 16
NW = NC * NS

E_SPLIT = 163840
GCHUNK = 80
SCHUNK = 80
HALF = HIDDEN // NC
NPASS = 2
NHALF = N_NODES // NPASS
ACC_ROWS = NHALF + 8
ZCHUNK = 16
OCHUNK = 40

_sc_mesh = plsc.VectorSubcoreMesh(core_axis_name="c", subcore_axis_name="s")


def _make_gather(e_cnt):
    e_per_w = e_cnt // NW
    giters = e_per_w // GCHUNK

    @functools.partial(
        pl.kernel,
        mesh=_sc_mesh,
        out_type=[
            jax.ShapeDtypeStruct((e_cnt, D_FEAT), jnp.float32),
            jax.ShapeDtypeStruct((e_cnt, D_FEAT), jnp.float32),
        ],
        scratch_types=[
            pltpu.VMEM((e_per_w,), jnp.int32),
            pltpu.VMEM((e_per_w,), jnp.int32),
            pltpu.VMEM((GCHUNK, D_FEAT), jnp.float32),
            pltpu.VMEM((GCHUNK, D_FEAT), jnp.float32),
            pltpu.VMEM((GCHUNK, D_FEAT), jnp.float32),
            pltpu.VMEM((GCHUNK, D_FEAT), jnp.float32),
            pltpu.SemaphoreType.DMA,
            pltpu.SemaphoreType.DMA,
            pltpu.SemaphoreType.DMA,
            pltpu.SemaphoreType.DMA,
        ],
    )
    def gather(h_hbm, row_hbm, col_hbm, src_hbm, tgt_hbm,
               rowi, coli, bufa0, bufb0, bufa1, bufb1, sa0, sb0, sa1, sb1):
        c = lax.axis_index("c")
        s = lax.axis_index("s")
        wid = s * NC + c
        base = wid * e_per_w
        pltpu.sync_copy(row_hbm.at[pl.ds(base, e_per_w)], rowi)
        pltpu.sync_copy(col_hbm.at[pl.ds(base, e_per_w)], coli)

        def issue(off, ba, bb, sema, semb):
            pltpu.async_copy(h_hbm.at[rowi.at[pl.ds(off, GCHUNK)]], ba, sema)
            pltpu.async_copy(h_hbm.at[coli.at[pl.ds(off, GCHUNK)]], bb, semb)

        def drain(ba, bb, sema, semb):
            pltpu.make_async_copy(h_hbm.at[pl.ds(0, GCHUNK)], ba, sema).wait()
            pltpu.make_async_copy(h_hbm.at[pl.ds(0, GCHUNK)], bb, semb).wait()

        def write(off, ba, bb):
            pltpu.sync_copy(ba, src_hbm.at[pl.ds(base + off, GCHUNK)])
            pltpu.sync_copy(bb, tgt_hbm.at[pl.ds(base + off, GCHUNK)])

        issue(0, bufa0, bufb0, sa0, sb0)

        def body(t, carry):
            off0 = 2 * t * GCHUNK
            off1 = off0 + GCHUNK
            issue(off1, bufa1, bufb1, sa1, sb1)
            drain(bufa0, bufb0, sa0, sb0)
            write(off0, bufa0, bufb0)

            @pl.when(2 * t + 2 < giters)
            def _():
                issue(off1 + GCHUNK, bufa0, bufb0, sa0, sb0)

            drain(bufa1, bufb1, sa1, sb1)
            write(off1, bufa1, bufb1)
            return carry

        lax.fori_loop(0, giters // 2, body, 0)
        if giters % 2:
            off = (giters - 1) * GCHUNK
            drain(bufa0, bufb0, sa0, sb0)
            write(off, bufa0, bufb0)

    return gather


_gather_a = _make_gather(E_SPLIT)
_gather_b = _make_gather(N_EDGES - E_SPLIT)


def _make_scatter(e_cnt):
    e_per_t = e_cnt // NS
    siters = e_per_t // SCHUNK

    @functools.partial(
        pl.kernel,
        mesh=_sc_mesh,
        out_type=jax.ShapeDtypeStruct((N_NODES, HIDDEN), jnp.float32),
        scratch_types=[
            pltpu.VMEM((siters, SCHUNK), jnp.int32),
            pltpu.VMEM((1, SCHUNK), jnp.int32),
            pltpu.VMEM((1, SCHUNK), jnp.int32),
            pltpu.VMEM((SCHUNK, HALF), jnp.float32),
            pltpu.VMEM((SCHUNK, HALF), jnp.float32),
            pltpu.VMEM((ZCHUNK, HALF), jnp.float32),
            pltpu.VMEM((OCHUNK, HALF), jnp.float32),
            pltpu.VMEM_SHARED((ACC_ROWS, HALF), jnp.float32),
            pltpu.SemaphoreType.DMA,
            pltpu.SemaphoreType.DMA,
            pltpu.SemaphoreType.DMA,
            pltpu.SemaphoreType.DMA,
        ],
    )
    def scatter(mij_hbm, row3_hbm, agg_hbm, idxv, lidx0, lidx1,
                bufv0, bufv1, zbuf, obuf, acc, sr0, sr1, ss0, ss1):
        c = lax.axis_index("c")
        s = lax.axis_index("s")

        zero = jnp.zeros((16,), jnp.float32)

        def zrow(i, carry):
            for q in range(HALF // 16):
                zbuf[i, pl.ds(q * 16, 16)] = zero
            return carry

        lax.fori_loop(0, ZCHUNK, zrow, 0)

        pltpu.sync_copy(row3_hbm.at[s], idxv)

        ziters = jnp.where(s == NS - 1, 13, 20)
        oiters = jnp.where(s == NS - 1, 5, 8)
        zstart = s * 320

        for p in range(NPASS):
            nbase = p * NHALF

            def zcp(t, carry):
                pltpu.sync_copy(zbuf, acc.at[pl.ds(zstart + t * ZCHUNK, ZCHUNK)])
                return carry

            lax.fori_loop(0, ziters, zcp, 0)
            plsc.subcore_barrier()

            def rd_issue(j, buf, sem):
                ebase = s * e_per_t + j * SCHUNK
                pltpu.async_copy(
                    mij_hbm.at[pl.ds(ebase, SCHUNK), pl.ds(c * HALF, HALF)],
                    buf, sem)

            def rd_wait(buf, sem):
                pltpu.make_async_copy(
                    mij_hbm.at[pl.ds(0, SCHUNK), pl.ds(0, HALF)], buf,
                    sem).wait()

            def remap(j, lidx):
                for k in range(SCHUNK // 16):
                    v = idxv[j, pl.ds(k * 16, 16)] - nbase
                    ok = (v >= 0) & (v < NHALF)
                    lidx[0, pl.ds(k * 16, 16)] = jnp.where(ok, v, NHALF)

            def sc_wait(buf, sem):
                pltpu.make_async_copy(buf, acc.at[pl.ds(0, SCHUNK)], sem).wait()

            rd_issue(0, bufv0, sr0)

            def ebody(t, carry):
                j0 = 2 * t
                j1 = j0 + 1

                @pl.when(t > 0)
                def _():
                    sc_wait(bufv1, ss1)

                rd_issue(j1, bufv1, sr1)
                rd_wait(bufv0, sr0)
                remap(j0, lidx0)
                pltpu.async_copy(bufv0, acc.at[lidx0.at[0]], ss0, add=True)
                rd_wait(bufv1, sr1)
                remap(j1, lidx1)
                pltpu.async_copy(bufv1, acc.at[lidx1.at[0]], ss1, add=True)
                sc_wait(bufv0, ss0)

                @pl.when(t < siters // 2 - 1)
                def _():
                    rd_issue(j0 + 2, bufv0, sr0)

                return carry

            lax.fori_loop(0, siters // 2, ebody, 0)
            sc_wait(bufv1, ss1)
            plsc.subcore_barrier()

            def obody(t, carry):
                rbase = zstart + t * OCHUNK
                pltpu.sync_copy(acc.at[pl.ds(rbase, OCHUNK)], obuf)
                pltpu.sync_copy(
                    obuf,
                    agg_hbm.at[pl.ds(nbase + rbase, OCHUNK),
                               pl.ds(c * HALF, HALF)])
                return carry

            lax.fori_loop(0, oiters, obody, 0)

    return scatter


_scatter_a = _make_scatter(E_SPLIT)
_scatter_b = _make_scatter(N_EDGES - E_SPLIT)


BE = 2560


def _edge_mlp_body(src_ref, tgt_ref, ea_ref, w1a, w1b, w1c, b1r, w2r, b2r,
                   out_ref):
    x = (jnp.dot(src_ref[...], w1a[...], preferred_element_type=jnp.float32)
         + jnp.dot(tgt_ref[...], w1b[...], preferred_element_type=jnp.float32)
         + jnp.dot(ea_ref[...], w1c[...], preferred_element_type=jnp.float32)
         + b1r[...])
    x = x * jax.nn.sigmoid(x)
    y = jnp.dot(x, w2r[...], preferred_element_type=jnp.float32) + b2r[...]
    out_ref[...] = y * jax.nn.sigmoid(y)


def _edge_mlp(src, tgt, ea, w1a, w1b, w1c, b1, w2, b2):
    e_cnt = src.shape[0]
    grid = (e_cnt // BE,)
    full = lambda shape: pl.BlockSpec(shape, lambda i: (0, 0))
    return pl.pallas_call(
        _edge_mlp_body,
        grid=grid,
        in_specs=[
            pl.BlockSpec((BE, D_FEAT), lambda i: (i, 0)),
            pl.BlockSpec((BE, D_FEAT), lambda i: (i, 0)),
            pl.BlockSpec((BE, D_EDGE), lambda i: (i, 0)),
            full((D_FEAT, HIDDEN)),
            full((D_FEAT, HIDDEN)),
            full((D_EDGE, HIDDEN)),
            full((1, HIDDEN)),
            full((HIDDEN, HIDDEN)),
            full((1, HIDDEN)),
        ],
        out_specs=pl.BlockSpec((BE, HIDDEN), lambda i: (i, 0)),
        out_shape=jax.ShapeDtypeStruct((e_cnt, HIDDEN), jnp.float32),
    )(src, tgt, ea, w1a, w1b, w1c, b1, w2, b2)


BN = 2000


def _node_mlp_body(h_ref, agga_ref, aggb_ref, w3a, w3b, b3r, w4r, b4r,
                   out_ref):
    agg = agga_ref[...] + aggb_ref[...]
    x = (jnp.dot(h_ref[...], w3a[...], preferred_element_type=jnp.float32)
         + jnp.dot(agg, w3b[...], preferred_element_type=jnp.float32)
         + b3r[...])
    x = x * jax.nn.sigmoid(x)
    out_ref[...] = (h_ref[...]
                    + jnp.dot(x, w4r[...], preferred_element_type=jnp.float32)
                    + b4r[...])


def _node_mlp(h, agga, aggb, w3a, w3b, b3, w4, b4):
    grid = (N_NODES // BN,)
    full = lambda shape: pl.BlockSpec(shape, lambda i: (0, 0))
    return pl.pallas_call(
        _node_mlp_body,
        grid=grid,
        in_specs=[
            pl.BlockSpec((BN, D_FEAT), lambda i: (i, 0)),
            pl.BlockSpec((BN, HIDDEN), lambda i: (i, 0)),
            pl.BlockSpec((BN, HIDDEN), lambda i: (i, 0)),
            full((D_FEAT, HIDDEN)),
            full((HIDDEN, HIDDEN)),
            full((1, HIDDEN)),
            full((HIDDEN, D_FEAT)),
            full((1, D_FEAT)),
        ],
        out_specs=pl.BlockSpec((BN, D_FEAT), lambda i: (i, 0)),
        out_shape=jax.ShapeDtypeStruct((N_NODES, D_FEAT), jnp.float32),
    )(h, agga, aggb, w3a, w3b, b3, w4, b4)


def kernel(h, edge_index, edge_attr, W1, b1, W2, b2, W3, b3, W4, b4):
    row = edge_index[0].astype(jnp.int32)
    col = edge_index[1].astype(jnp.int32)

    w1a, w1b, w1c = W1[:D_FEAT], W1[D_FEAT:2 * D_FEAT], W1[2 * D_FEAT:]
    b1r, b2r, b3r, b4r = (b.reshape(1, -1) for b in (b1, b2, b3, b4))

    ecnt_b = N_EDGES - E_SPLIT
    row_a, row_b = row[:E_SPLIT], row[E_SPLIT:]
    col_a, col_b = col[:E_SPLIT], col[E_SPLIT:]
    ea_a, ea_b = edge_attr[:E_SPLIT], edge_attr[E_SPLIT:]

    src_a, tgt_a = _gather_a(h, row_a, col_a)
    src_b, tgt_b = _gather_b(h, row_b, col_b)

    mij_a = _edge_mlp(src_a, tgt_a, ea_a, w1a, w1b, w1c, b1r, W2, b2r)
    mij_b = _edge_mlp(src_b, tgt_b, ea_b, w1a, w1b, w1c, b1r, W2, b2r)

    row3_a = row_a.reshape(NS, E_SPLIT // NS // SCHUNK, SCHUNK)
    row3_b = row_b.reshape(NS, ecnt_b // NS // SCHUNK, SCHUNK)
    agg_a = _scatter_a(mij_a, row3_a)
    agg_b = _scatter_b(mij_b, row3_b)

    h_new = _node_mlp(h, agg_a, agg_b, W3[:D_FEAT], W3[D_FEAT:],
                      b3r, W4, b4r)
    mij = jnp.concatenate([mij_a, mij_b], axis=0)
    return (h_new, mij)


# speedup vs baseline: 3.0451x; 1.0436x over previous
import functools

import jax
import jax.numpy as jnp
from jax import lax
from jax.experimental import pallas as pl
from jax.experimental.pallas import tpu as pltpu
from jax.experimental.pallas import tpu_sc as plsc

N_NODES = 10000
N_EDGES = 320000
D_FEAT = 128
D_EDGE = 16
HIDDEN = 256

NC = 2
NS = 16
NW = NC * NS

E_SPLIT = 163840
HALF = HIDDEN // NC
NPASS = 2
NHALF = N_NODES // NPASS
ACC_ROWS = NHALF + 8
ZCHUNK = 16
OCHUNK = 40

_sc_mesh = plsc.VectorSubcoreMesh(core_axis_name="c", subcore_axis_name="s")


def _make_gather(e_cnt, GCHUNK):
    e_per_w = e_cnt // NW
    giters = e_per_w // GCHUNK

    @functools.partial(
        pl.kernel,
        mesh=_sc_mesh,
        out_type=[
            jax.ShapeDtypeStruct((e_cnt, D_FEAT), jnp.float32),
            jax.ShapeDtypeStruct((e_cnt, D_FEAT), jnp.float32),
        ],
        scratch_types=[
            pltpu.VMEM((e_per_w,), jnp.int32),
            pltpu.VMEM((e_per_w,), jnp.int32),
            pltpu.VMEM((GCHUNK, D_FEAT), jnp.float32),
            pltpu.VMEM((GCHUNK, D_FEAT), jnp.float32),
            pltpu.VMEM((GCHUNK, D_FEAT), jnp.float32),
            pltpu.VMEM((GCHUNK, D_FEAT), jnp.float32),
            pltpu.SemaphoreType.DMA,
            pltpu.SemaphoreType.DMA,
            pltpu.SemaphoreType.DMA,
            pltpu.SemaphoreType.DMA,
        ],
    )
    def gather(h_hbm, row_hbm, col_hbm, src_hbm, tgt_hbm,
               rowi, coli, bufa0, bufb0, bufa1, bufb1, sa0, sb0, sa1, sb1):
        c = lax.axis_index("c")
        s = lax.axis_index("s")
        wid = s * NC + c
        base = wid * e_per_w
        pltpu.sync_copy(row_hbm.at[pl.ds(base, e_per_w)], rowi)
        pltpu.sync_copy(col_hbm.at[pl.ds(base, e_per_w)], coli)

        def issue(off, ba, bb, sema, semb):
            pltpu.async_copy(h_hbm.at[rowi.at[pl.ds(off, GCHUNK)]], ba, sema)
            pltpu.async_copy(h_hbm.at[coli.at[pl.ds(off, GCHUNK)]], bb, semb)

        def drain(ba, bb, sema, semb):
            pltpu.make_async_copy(h_hbm.at[pl.ds(0, GCHUNK)], ba, sema).wait()
            pltpu.make_async_copy(h_hbm.at[pl.ds(0, GCHUNK)], bb, semb).wait()

        def write(off, ba, bb):
            pltpu.sync_copy(ba, src_hbm.at[pl.ds(base + off, GCHUNK)])
            pltpu.sync_copy(bb, tgt_hbm.at[pl.ds(base + off, GCHUNK)])

        issue(0, bufa0, bufb0, sa0, sb0)

        def body(t, carry):
            off0 = 2 * t * GCHUNK
            off1 = off0 + GCHUNK
            issue(off1, bufa1, bufb1, sa1, sb1)
            drain(bufa0, bufb0, sa0, sb0)
            write(off0, bufa0, bufb0)

            @pl.when(2 * t + 2 < giters)
            def _():
                issue(off1 + GCHUNK, bufa0, bufb0, sa0, sb0)

            drain(bufa1, bufb1, sa1, sb1)
            write(off1, bufa1, bufb1)
            return carry

        lax.fori_loop(0, giters // 2, body, 0)
        if giters % 2:
            off = (giters - 1) * GCHUNK
            drain(bufa0, bufb0, sa0, sb0)
            write(off, bufa0, bufb0)

    return gather


_gather_a = _make_gather(E_SPLIT, 128)
_gather_b = _make_gather(N_EDGES - E_SPLIT, 80)


def _make_scatter(e_cnt, SCHUNK):
    e_per_t = e_cnt // NS
    siters = e_per_t // SCHUNK

    @functools.partial(
        pl.kernel,
        mesh=_sc_mesh,
        out_type=jax.ShapeDtypeStruct((N_NODES, HIDDEN), jnp.float32),
        scratch_types=[
            pltpu.VMEM((siters, SCHUNK), jnp.int32),
            pltpu.VMEM((1, SCHUNK), jnp.int32),
            pltpu.VMEM((1, SCHUNK), jnp.int32),
            pltpu.VMEM((SCHUNK, HALF), jnp.float32),
            pltpu.VMEM((SCHUNK, HALF), jnp.float32),
            pltpu.VMEM((ZCHUNK, HALF), jnp.float32),
            pltpu.VMEM((OCHUNK, HALF), jnp.float32),
            pltpu.VMEM_SHARED((ACC_ROWS, HALF), jnp.float32),
            pltpu.SemaphoreType.DMA,
            pltpu.SemaphoreType.DMA,
            pltpu.SemaphoreType.DMA,
            pltpu.SemaphoreType.DMA,
        ],
    )
    def scatter(mij_hbm, row3_hbm, agg_hbm, idxv, lidx0, lidx1,
                bufv0, bufv1, zbuf, obuf, acc, sr0, sr1, ss0, ss1):
        c = lax.axis_index("c")
        s = lax.axis_index("s")

        zero = jnp.zeros((16,), jnp.float32)

        def zrow(i, carry):
            for q in range(HALF // 16):
                zbuf[i, pl.ds(q * 16, 16)] = zero
            return carry

        lax.fori_loop(0, ZCHUNK, zrow, 0)

        pltpu.sync_copy(row3_hbm.at[s], idxv)

        ziters = jnp.where(s == NS - 1, 13, 20)
        oiters = jnp.where(s == NS - 1, 5, 8)
        zstart = s * 320

        for p in range(NPASS):
            nbase = p * NHALF

            def zcp(t, carry):
                pltpu.sync_copy(zbuf, acc.at[pl.ds(zstart + t * ZCHUNK, ZCHUNK)])
                return carry

            lax.fori_loop(0, ziters, zcp, 0)
            plsc.subcore_barrier()

            def rd_issue(j, buf, sem):
                ebase = s * e_per_t + j * SCHUNK
                pltpu.async_copy(
                    mij_hbm.at[pl.ds(ebase, SCHUNK), pl.ds(c * HALF, HALF)],
                    buf, sem)

            def rd_wait(buf, sem):
                pltpu.make_async_copy(
                    mij_hbm.at[pl.ds(0, SCHUNK), pl.ds(0, HALF)], buf,
                    sem).wait()

            trash = NHALF + (lax.iota(jnp.int32, 16) & 7)

            def remap(j, lidx):
                for k in range(SCHUNK // 16):
                    v = idxv[j, pl.ds(k * 16, 16)] - nbase
                    ok = (v >= 0) & (v < NHALF)
                    lidx[0, pl.ds(k * 16, 16)] = jnp.where(ok, v, trash)

            def sc_wait(buf, sem):
                pltpu.make_async_copy(buf, acc.at[pl.ds(0, SCHUNK)], sem).wait()

            rd_issue(0, bufv0, sr0)

            def ebody(t, carry):
                j0 = 2 * t
                j1 = j0 + 1

                @pl.when(t > 0)
                def _():
                    sc_wait(bufv1, ss1)

                rd_issue(j1, bufv1, sr1)
                rd_wait(bufv0, sr0)
                remap(j0, lidx0)
                pltpu.async_copy(bufv0, acc.at[lidx0.at[0]], ss0, add=True)
                rd_wait(bufv1, sr1)
                remap(j1, lidx1)
                pltpu.async_copy(bufv1, acc.at[lidx1.at[0]], ss1, add=True)
                sc_wait(bufv0, ss0)

                @pl.when(t < siters // 2 - 1)
                def _():
                    rd_issue(j0 + 2, bufv0, sr0)

                return carry

            lax.fori_loop(0, siters // 2, ebody, 0)
            sc_wait(bufv1, ss1)
            plsc.subcore_barrier()

            def obody(t, carry):
                rbase = zstart + t * OCHUNK
                pltpu.sync_copy(acc.at[pl.ds(rbase, OCHUNK)], obuf)
                pltpu.sync_copy(
                    obuf,
                    agg_hbm.at[pl.ds(nbase + rbase, OCHUNK),
                               pl.ds(c * HALF, HALF)])
                return carry

            lax.fori_loop(0, oiters, obody, 0)

    return scatter


_scatter_a = _make_scatter(E_SPLIT, 128)
_scatter_b = _make_scatter(N_EDGES - E_SPLIT, 80)


BE = 2560


def _edge_mlp_body(src_ref, tgt_ref, ea_ref, w1a, w1b, w1c, b1r, w2r, b2r,
                   out_ref):
    x = (jnp.dot(src_ref[...], w1a[...], preferred_element_type=jnp.float32)
         + jnp.dot(tgt_ref[...], w1b[...], preferred_element_type=jnp.float32)
         + jnp.dot(ea_ref[...], w1c[...], preferred_element_type=jnp.float32)
         + b1r[...])
    x = x * jax.nn.sigmoid(x)
    y = jnp.dot(x, w2r[...], preferred_element_type=jnp.float32) + b2r[...]
    out_ref[...] = y * jax.nn.sigmoid(y)


def _edge_mlp(src, tgt, ea, w1a, w1b, w1c, b1, w2, b2):
    e_cnt = src.shape[0]
    grid = (e_cnt // BE,)
    full = lambda shape: pl.BlockSpec(shape, lambda i: (0, 0))
    return pl.pallas_call(
        _edge_mlp_body,
        grid=grid,
        in_specs=[
            pl.BlockSpec((BE, D_FEAT), lambda i: (i, 0)),
            pl.BlockSpec((BE, D_FEAT), lambda i: (i, 0)),
            pl.BlockSpec((BE, D_EDGE), lambda i: (i, 0)),
            full((D_FEAT, HIDDEN)),
            full((D_FEAT, HIDDEN)),
            full((D_EDGE, HIDDEN)),
            full((1, HIDDEN)),
            full((HIDDEN, HIDDEN)),
            full((1, HIDDEN)),
        ],
        out_specs=pl.BlockSpec((BE, HIDDEN), lambda i: (i, 0)),
        out_shape=jax.ShapeDtypeStruct((e_cnt, HIDDEN), jnp.float32),
    )(src, tgt, ea, w1a, w1b, w1c, b1, w2, b2)


BN = 2000


def _node_mlp_body(h_ref, agga_ref, aggb_ref, w3a, w3b, b3r, w4r, b4r,
                   out_ref):
    agg = agga_ref[...] + aggb_ref[...]
    x = (jnp.dot(h_ref[...], w3a[...], preferred_element_type=jnp.float32)
         + jnp.dot(agg, w3b[...], preferred_element_type=jnp.float32)
         + b3r[...])
    x = x * jax.nn.sigmoid(x)
    out_ref[...] = (h_ref[...]
                    + jnp.dot(x, w4r[...], preferred_element_type=jnp.float32)
                    + b4r[...])


def _node_mlp(h, agga, aggb, w3a, w3b, b3, w4, b4):
    grid = (N_NODES // BN,)
    full = lambda shape: pl.BlockSpec(shape, lambda i: (0, 0))
    return pl.pallas_call(
        _node_mlp_body,
        grid=grid,
        in_specs=[
            pl.BlockSpec((BN, D_FEAT), lambda i: (i, 0)),
            pl.BlockSpec((BN, HIDDEN), lambda i: (i, 0)),
            pl.BlockSpec((BN, HIDDEN), lambda i: (i, 0)),
            full((D_FEAT, HIDDEN)),
            full((HIDDEN, HIDDEN)),
            full((1, HIDDEN)),
            full((HIDDEN, D_FEAT)),
            full((1, D_FEAT)),
        ],
        out_specs=pl.BlockSpec((BN, D_FEAT), lambda i: (i, 0)),
        out_shape=jax.ShapeDtypeStruct((N_NODES, D_FEAT), jnp.float32),
    )(h, agga, aggb, w3a, w3b, b3, w4, b4)


def kernel(h, edge_index, edge_attr, W1, b1, W2, b2, W3, b3, W4, b4):
    row = edge_index[0].astype(jnp.int32)
    col = edge_index[1].astype(jnp.int32)

    w1a, w1b, w1c = W1[:D_FEAT], W1[D_FEAT:2 * D_FEAT], W1[2 * D_FEAT:]
    b1r, b2r, b3r, b4r = (b.reshape(1, -1) for b in (b1, b2, b3, b4))

    ecnt_b = N_EDGES - E_SPLIT
    row_a, row_b = row[:E_SPLIT], row[E_SPLIT:]
    col_a, col_b = col[:E_SPLIT], col[E_SPLIT:]
    ea_a, ea_b = edge_attr[:E_SPLIT], edge_attr[E_SPLIT:]

    src_a, tgt_a = _gather_a(h, row_a, col_a)
    src_b, tgt_b = _gather_b(h, row_b, col_b)

    mij_a = _edge_mlp(src_a, tgt_a, ea_a, w1a, w1b, w1c, b1r, W2, b2r)
    mij_b = _edge_mlp(src_b, tgt_b, ea_b, w1a, w1b, w1c, b1r, W2, b2r)

    row3_a = row_a.reshape(NS, E_SPLIT // NS // 128, 128)
    row3_b = row_b.reshape(NS, ecnt_b // NS // 80, 80)
    agg_a = _scatter_a(mij_a, row3_a)
    agg_b = _scatter_b(mij_b, row3_b)

    h_new = _node_mlp(h, agg_a, agg_b, W3[:D_FEAT], W3[D_FEAT:],
                      b3r, W4, b4r)
    mij = jnp.concatenate([mij_a, mij_b], axis=0)
    return (h_new, mij)


# speedup vs baseline: 3.1615x; 1.0382x over previous
import functools

import jax
import jax.numpy as jnp
from jax import lax
from jax.experimental import pallas as pl
from jax.experimental.pallas import tpu as pltpu
from jax.experimental.pallas import tpu_sc as plsc

N_NODES = 10000
N_EDGES = 320000
D_FEAT = 128
D_EDGE = 16
HIDDEN = 256

NC = 2
NS = 16
NW = NC * NS

E_SPLIT = 163840
HALF = HIDDEN // NC
NPASS = 2
NHALF = N_NODES // NPASS
ACC_ROWS = NHALF + 8
ZCHUNK = 16
OCHUNK = 40

_sc_mesh = plsc.VectorSubcoreMesh(core_axis_name="c", subcore_axis_name="s")


def _make_gather_ring(e_cnt, GCHUNK, RING):
    e_per_w = e_cnt // NW
    giters = e_per_w // GCHUNK
    assert giters % RING == 0

    scratch = [pltpu.VMEM((e_per_w,), jnp.int32),
               pltpu.VMEM((e_per_w,), jnp.int32)]
    scratch += [pltpu.VMEM((GCHUNK, D_FEAT), jnp.float32)] * (2 * RING)
    scratch += [pltpu.SemaphoreType.DMA] * (4 * RING)

    @functools.partial(
        pl.kernel,
        mesh=_sc_mesh,
        out_type=[
            jax.ShapeDtypeStruct((e_cnt, D_FEAT), jnp.float32),
            jax.ShapeDtypeStruct((e_cnt, D_FEAT), jnp.float32),
        ],
        scratch_types=scratch,
    )
    def gather(h_hbm, row_hbm, col_hbm, src_hbm, tgt_hbm, *scr):
        rowi, coli = scr[0], scr[1]
        bufa = scr[2:2 + RING]
        bufb = scr[2 + RING:2 + 2 * RING]
        sga = scr[2 + 2 * RING:2 + 3 * RING]
        sgb = scr[2 + 3 * RING:2 + 4 * RING]
        swa = scr[2 + 4 * RING:2 + 5 * RING]
        swb = scr[2 + 5 * RING:2 + 6 * RING]

        c = lax.axis_index("c")
        s = lax.axis_index("s")
        wid = s * NC + c
        base = wid * e_per_w
        pltpu.sync_copy(row_hbm.at[pl.ds(base, e_per_w)], rowi)
        pltpu.sync_copy(col_hbm.at[pl.ds(base, e_per_w)], coli)

        def g_issue(j, k):
            off = j * GCHUNK
            pltpu.async_copy(h_hbm.at[rowi.at[pl.ds(off, GCHUNK)]],
                             bufa[k], sga[k])
            pltpu.async_copy(h_hbm.at[coli.at[pl.ds(off, GCHUNK)]],
                             bufb[k], sgb[k])

        def g_wait(k):
            pltpu.make_async_copy(h_hbm.at[pl.ds(0, GCHUNK)], bufa[k],
                                  sga[k]).wait()
            pltpu.make_async_copy(h_hbm.at[pl.ds(0, GCHUNK)], bufb[k],
                                  sgb[k]).wait()

        def w_issue(j, k):
            off = j * GCHUNK
            pltpu.async_copy(bufa[k], src_hbm.at[pl.ds(base + off, GCHUNK)],
                             swa[k])
            pltpu.async_copy(bufb[k], tgt_hbm.at[pl.ds(base + off, GCHUNK)],
                             swb[k])

        def w_wait(k):
            pltpu.make_async_copy(bufa[k], src_hbm.at[pl.ds(0, GCHUNK)],
                                  swa[k]).wait()
            pltpu.make_async_copy(bufb[k], tgt_hbm.at[pl.ds(0, GCHUNK)],
                                  swb[k]).wait()

        g_issue(0, 0)
        g_issue(1, 1)

        def body(t, carry):
            for k in range(RING):
                j = RING * t + k
                g_wait(k)
                w_issue(j, k)
                k2 = (k + 2) % RING
                if k < RING - 2:
                    @pl.when(t > 0)
                    def _():
                        w_wait(k2)

                    g_issue(j + 2, k2)
                else:
                    w_wait(k2)

                    @pl.when(t < giters // RING - 1)
                    def _():
                        g_issue(j + 2, k2)
            return carry

        lax.fori_loop(0, giters // RING, body, 0)
        for jj in range(giters - RING + 2, giters):
            w_wait(jj % RING)

    return gather


def _make_gather(e_cnt, GCHUNK):
    e_per_w = e_cnt // NW
    giters = e_per_w // GCHUNK

    @functools.partial(
        pl.kernel,
        mesh=_sc_mesh,
        out_type=[
            jax.ShapeDtypeStruct((e_cnt, D_FEAT), jnp.float32),
            jax.ShapeDtypeStruct((e_cnt, D_FEAT), jnp.float32),
        ],
        scratch_types=[
            pltpu.VMEM((e_per_w,), jnp.int32),
            pltpu.VMEM((e_per_w,), jnp.int32),
            pltpu.VMEM((GCHUNK, D_FEAT), jnp.float32),
            pltpu.VMEM((GCHUNK, D_FEAT), jnp.float32),
            pltpu.VMEM((GCHUNK, D_FEAT), jnp.float32),
            pltpu.VMEM((GCHUNK, D_FEAT), jnp.float32),
            pltpu.SemaphoreType.DMA,
            pltpu.SemaphoreType.DMA,
            pltpu.SemaphoreType.DMA,
            pltpu.SemaphoreType.DMA,
        ],
    )
    def gather(h_hbm, row_hbm, col_hbm, src_hbm, tgt_hbm,
               rowi, coli, bufa0, bufb0, bufa1, bufb1, sa0, sb0, sa1, sb1):
        c = lax.axis_index("c")
        s = lax.axis_index("s")
        wid = s * NC + c
        base = wid * e_per_w
        pltpu.sync_copy(row_hbm.at[pl.ds(base, e_per_w)], rowi)
        pltpu.sync_copy(col_hbm.at[pl.ds(base, e_per_w)], coli)

        def issue(off, ba, bb, sema, semb):
            pltpu.async_copy(h_hbm.at[rowi.at[pl.ds(off, GCHUNK)]], ba, sema)
            pltpu.async_copy(h_hbm.at[coli.at[pl.ds(off, GCHUNK)]], bb, semb)

        def drain(ba, bb, sema, semb):
            pltpu.make_async_copy(h_hbm.at[pl.ds(0, GCHUNK)], ba, sema).wait()
            pltpu.make_async_copy(h_hbm.at[pl.ds(0, GCHUNK)], bb, semb).wait()

        def write(off, ba, bb):
            pltpu.sync_copy(ba, src_hbm.at[pl.ds(base + off, GCHUNK)])
            pltpu.sync_copy(bb, tgt_hbm.at[pl.ds(base + off, GCHUNK)])

        issue(0, bufa0, bufb0, sa0, sb0)

        def body(t, carry):
            off0 = 2 * t * GCHUNK
            off1 = off0 + GCHUNK
            issue(off1, bufa1, bufb1, sa1, sb1)
            drain(bufa0, bufb0, sa0, sb0)
            write(off0, bufa0, bufb0)

            @pl.when(2 * t + 2 < giters)
            def _():
                issue(off1 + GCHUNK, bufa0, bufb0, sa0, sb0)

            drain(bufa1, bufb1, sa1, sb1)
            write(off1, bufa1, bufb1)
            return carry

        lax.fori_loop(0, giters // 2, body, 0)
        if giters % 2:
            off = (giters - 1) * GCHUNK
            drain(bufa0, bufb0, sa0, sb0)
            write(off, bufa0, bufb0)

    return gather


_gather_a = _make_gather_ring(E_SPLIT, 64, 4)
_gather_b = _make_gather(N_EDGES - E_SPLIT, 80)


def _make_scatter_ring(e_cnt, SCHUNK, RING):
    e_per_t = e_cnt // NS
    siters = e_per_t // SCHUNK
    assert siters % RING == 0

    scratch = [pltpu.VMEM((siters, SCHUNK), jnp.int32)]
    scratch += [pltpu.VMEM((1, SCHUNK), jnp.int32)] * RING
    scratch += [pltpu.VMEM((SCHUNK, HALF), jnp.float32)] * RING
    scratch += [pltpu.VMEM((ZCHUNK, HALF), jnp.float32),
                pltpu.VMEM((OCHUNK, HALF), jnp.float32),
                pltpu.VMEM_SHARED((ACC_ROWS, HALF), jnp.float32)]
    scratch += [pltpu.SemaphoreType.DMA] * (2 * RING)

    @functools.partial(
        pl.kernel,
        mesh=_sc_mesh,
        out_type=jax.ShapeDtypeStruct((N_NODES, HIDDEN), jnp.float32),
        scratch_types=scratch,
    )
    def scatter(mij_hbm, row3_hbm, agg_hbm, *scr):
        idxv = scr[0]
        lidx = scr[1:1 + RING]
        bufv = scr[1 + RING:1 + 2 * RING]
        zbuf, obuf, acc = scr[1 + 2 * RING:4 + 2 * RING]
        sr = scr[4 + 2 * RING:4 + 3 * RING]
        ss = scr[4 + 3 * RING:4 + 4 * RING]

        c = lax.axis_index("c")
        s = lax.axis_index("s")

        zero = jnp.zeros((16,), jnp.float32)

        def zrow(i, carry):
            for q in range(HALF // 16):
                zbuf[i, pl.ds(q * 16, 16)] = zero
            return carry

        lax.fori_loop(0, ZCHUNK, zrow, 0)
        pltpu.sync_copy(row3_hbm.at[s], idxv)

        ziters = jnp.where(s == NS - 1, 13, 20)
        oiters = jnp.where(s == NS - 1, 5, 8)
        zstart = s * 320

        for p in range(NPASS):
            nbase = p * NHALF

            def zcp(t, carry):
                pltpu.sync_copy(zbuf, acc.at[pl.ds(zstart + t * ZCHUNK, ZCHUNK)])
                return carry

            lax.fori_loop(0, ziters, zcp, 0)
            plsc.subcore_barrier()

            def rd_issue(j, k):
                ebase = s * e_per_t + j * SCHUNK
                pltpu.async_copy(
                    mij_hbm.at[pl.ds(ebase, SCHUNK), pl.ds(c * HALF, HALF)],
                    bufv[k], sr[k])

            def rd_wait(k):
                pltpu.make_async_copy(
                    mij_hbm.at[pl.ds(0, SCHUNK), pl.ds(0, HALF)], bufv[k],
                    sr[k]).wait()

            trash = NHALF + (lax.iota(jnp.int32, 16) & 7)

            def remap(j, k):
                for q in range(SCHUNK // 16):
                    v = idxv[j, pl.ds(q * 16, 16)] - nbase
                    ok = (v >= 0) & (v < NHALF)
                    lidx[k][0, pl.ds(q * 16, 16)] = jnp.where(ok, v, trash)

            def sc_wait(k):
                pltpu.make_async_copy(bufv[k], acc.at[pl.ds(0, SCHUNK)],
                                      ss[k]).wait()

            rd_issue(0, 0)
            rd_issue(1, 1)

            def ebody(t, carry):
                for k in range(RING):
                    j = RING * t + k
                    rd_wait(k)
                    remap(j, k)
                    pltpu.async_copy(bufv[k], acc.at[lidx[k].at[0]], ss[k],
                                     add=True)
                    k2 = (k + 2) % RING
                    if k < RING - 2:
                        @pl.when(t > 0)
                        def _():
                            sc_wait(k2)

                        rd_issue(j + 2, k2)
                    else:
                        sc_wait(k2)

                        @pl.when(t < siters // RING - 1)
                        def _():
                            rd_issue(j + 2, k2)
                return carry

            lax.fori_loop(0, siters // RING, ebody, 0)
            for jj in range(siters - RING + 2, siters):
                sc_wait(jj % RING)
            plsc.subcore_barrier()

            def obody(t, carry):
                rbase = zstart + t * OCHUNK
                pltpu.sync_copy(acc.at[pl.ds(rbase, OCHUNK)], obuf)
                pltpu.sync_copy(
                    obuf,
                    agg_hbm.at[pl.ds(nbase + rbase, OCHUNK),
                               pl.ds(c * HALF, HALF)])
                return carry

            lax.fori_loop(0, oiters, obody, 0)

    return scatter


def _make_scatter(e_cnt, SCHUNK):
    e_per_t = e_cnt // NS
    siters = e_per_t // SCHUNK

    @functools.partial(
        pl.kernel,
        mesh=_sc_mesh,
        out_type=jax.ShapeDtypeStruct((N_NODES, HIDDEN), jnp.float32),
        scratch_types=[
            pltpu.VMEM((siters, SCHUNK), jnp.int32),
            pltpu.VMEM((1, SCHUNK), jnp.int32),
            pltpu.VMEM((1, SCHUNK), jnp.int32),
            pltpu.VMEM((SCHUNK, HALF), jnp.float32),
            pltpu.VMEM((SCHUNK, HALF), jnp.float32),
            pltpu.VMEM((ZCHUNK, HALF), jnp.float32),
            pltpu.VMEM((OCHUNK, HALF), jnp.float32),
            pltpu.VMEM_SHARED((ACC_ROWS, HALF), jnp.float32),
            pltpu.SemaphoreType.DMA,
            pltpu.SemaphoreType.DMA,
            pltpu.SemaphoreType.DMA,
            pltpu.SemaphoreType.DMA,
        ],
    )
    def scatter(mij_hbm, row3_hbm, agg_hbm, idxv, lidx0, lidx1,
                bufv0, bufv1, zbuf, obuf, acc, sr0, sr1, ss0, ss1):
        c = lax.axis_index("c")
        s = lax.axis_index("s")

        zero = jnp.zeros((16,), jnp.float32)

        def zrow(i, carry):
            for q in range(HALF // 16):
                zbuf[i, pl.ds(q * 16, 16)] = zero
            return carry

        lax.fori_loop(0, ZCHUNK, zrow, 0)

        pltpu.sync_copy(row3_hbm.at[s], idxv)

        ziters = jnp.where(s == NS - 1, 13, 20)
        oiters = jnp.where(s == NS - 1, 5, 8)
        zstart = s * 320

        for p in range(NPASS):
            nbase = p * NHALF

            def zcp(t, carry):
                pltpu.sync_copy(zbuf, acc.at[pl.ds(zstart + t * ZCHUNK, ZCHUNK)])
                return carry

            lax.fori_loop(0, ziters, zcp, 0)
            plsc.subcore_barrier()

            def rd_issue(j, buf, sem):
                ebase = s * e_per_t + j * SCHUNK
                pltpu.async_copy(
                    mij_hbm.at[pl.ds(ebase, SCHUNK), pl.ds(c * HALF, HALF)],
                    buf, sem)

            def rd_wait(buf, sem):
                pltpu.make_async_copy(
                    mij_hbm.at[pl.ds(0, SCHUNK), pl.ds(0, HALF)], buf,
                    sem).wait()

            trash = NHALF + (lax.iota(jnp.int32, 16) & 7)

            def remap(j, lidx):
                for k in range(SCHUNK // 16):
                    v = idxv[j, pl.ds(k * 16, 16)] - nbase
                    ok = (v >= 0) & (v < NHALF)
                    lidx[0, pl.ds(k * 16, 16)] = jnp.where(ok, v, trash)

            def sc_wait(buf, sem):
                pltpu.make_async_copy(buf, acc.at[pl.ds(0, SCHUNK)], sem).wait()

            rd_issue(0, bufv0, sr0)

            def ebody(t, carry):
                j0 = 2 * t
                j1 = j0 + 1

                @pl.when(t > 0)
                def _():
                    sc_wait(bufv1, ss1)

                rd_issue(j1, bufv1, sr1)
                rd_wait(bufv0, sr0)
                remap(j0, lidx0)
                pltpu.async_copy(bufv0, acc.at[lidx0.at[0]], ss0, add=True)
                rd_wait(bufv1, sr1)
                remap(j1, lidx1)
                pltpu.async_copy(bufv1, acc.at[lidx1.at[0]], ss1, add=True)
                sc_wait(bufv0, ss0)

                @pl.when(t < siters // 2 - 1)
                def _():
                    rd_issue(j0 + 2, bufv0, sr0)

                return carry

            lax.fori_loop(0, siters // 2, ebody, 0)
            sc_wait(bufv1, ss1)
            plsc.subcore_barrier()

            def obody(t, carry):
                rbase = zstart + t * OCHUNK
                pltpu.sync_copy(acc.at[pl.ds(rbase, OCHUNK)], obuf)
                pltpu.sync_copy(
                    obuf,
                    agg_hbm.at[pl.ds(nbase + rbase, OCHUNK),
                               pl.ds(c * HALF, HALF)])
                return carry

            lax.fori_loop(0, oiters, obody, 0)

    return scatter


_scatter_a = _make_scatter_ring(E_SPLIT, 128, 4)
_scatter_b = _make_scatter(N_EDGES - E_SPLIT, 80)


BE = 2560


def _edge_mlp_body(src_ref, tgt_ref, ea_ref, w1a, w1b, w1c, b1r, w2r, b2r,
                   out_ref):
    x = (jnp.dot(src_ref[...], w1a[...], preferred_element_type=jnp.float32)
         + jnp.dot(tgt_ref[...], w1b[...], preferred_element_type=jnp.float32)
         + jnp.dot(ea_ref[...], w1c[...], preferred_element_type=jnp.float32)
         + b1r[...])
    x = x * jax.nn.sigmoid(x)
    y = jnp.dot(x, w2r[...], preferred_element_type=jnp.float32) + b2r[...]
    out_ref[...] = y * jax.nn.sigmoid(y)


def _edge_mlp(src, tgt, ea, w1a, w1b, w1c, b1, w2, b2):
    e_cnt = src.shape[0]
    grid = (e_cnt // BE,)
    full = lambda shape: pl.BlockSpec(shape, lambda i: (0, 0))
    return pl.pallas_call(
        _edge_mlp_body,
        grid=grid,
        in_specs=[
            pl.BlockSpec((BE, D_FEAT), lambda i: (i, 0)),
            pl.BlockSpec((BE, D_FEAT), lambda i: (i, 0)),
            pl.BlockSpec((BE, D_EDGE), lambda i: (i, 0)),
            full((D_FEAT, HIDDEN)),
            full((D_FEAT, HIDDEN)),
            full((D_EDGE, HIDDEN)),
            full((1, HIDDEN)),
            full((HIDDEN, HIDDEN)),
            full((1, HIDDEN)),
        ],
        out_specs=pl.BlockSpec((BE, HIDDEN), lambda i: (i, 0)),
        out_shape=jax.ShapeDtypeStruct((e_cnt, HIDDEN), jnp.float32),
    )(src, tgt, ea, w1a, w1b, w1c, b1, w2, b2)


BN = 2000


def _node_mlp_body(h_ref, agga_ref, aggb_ref, w3a, w3b, b3r, w4r, b4r,
                   out_ref):
    agg = agga_ref[...] + aggb_ref[...]
    x = (jnp.dot(h_ref[...], w3a[...], preferred_element_type=jnp.float32)
         + jnp.dot(agg, w3b[...], preferred_element_type=jnp.float32)
         + b3r[...])
    x = x * jax.nn.sigmoid(x)
    out_ref[...] = (h_ref[...]
                    + jnp.dot(x, w4r[...], preferred_element_type=jnp.float32)
                    + b4r[...])


def _node_mlp(h, agga, aggb, w3a, w3b, b3, w4, b4):
    grid = (N_NODES // BN,)
    full = lambda shape: pl.BlockSpec(shape, lambda i: (0, 0))
    return pl.pallas_call(
        _node_mlp_body,
        grid=grid,
        in_specs=[
            pl.BlockSpec((BN, D_FEAT), lambda i: (i, 0)),
            pl.BlockSpec((BN, HIDDEN), lambda i: (i, 0)),
            pl.BlockSpec((BN, HIDDEN), lambda i: (i, 0)),
            full((D_FEAT, HIDDEN)),
            full((HIDDEN, HIDDEN)),
            full((1, HIDDEN)),
            full((HIDDEN, D_FEAT)),
            full((1, D_FEAT)),
        ],
        out_specs=pl.BlockSpec((BN, D_FEAT), lambda i: (i, 0)),
        out_shape=jax.ShapeDtypeStruct((N_NODES, D_FEAT), jnp.float32),
    )(h, agga, aggb, w3a, w3b, b3, w4, b4)


def kernel(h, edge_index, edge_attr, W1, b1, W2, b2, W3, b3, W4, b4):
    row = edge_index[0].astype(jnp.int32)
    col = edge_index[1].astype(jnp.int32)

    w1a, w1b, w1c = W1[:D_FEAT], W1[D_FEAT:2 * D_FEAT], W1[2 * D_FEAT:]
    b1r, b2r, b3r, b4r = (b.reshape(1, -1) for b in (b1, b2, b3, b4))

    ecnt_b = N_EDGES - E_SPLIT
    row_a, row_b = row[:E_SPLIT], row[E_SPLIT:]
    col_a, col_b = col[:E_SPLIT], col[E_SPLIT:]
    ea_a, ea_b = edge_attr[:E_SPLIT], edge_attr[E_SPLIT:]

    src_a, tgt_a = _gather_a(h, row_a, col_a)
    src_b, tgt_b = _gather_b(h, row_b, col_b)

    mij_a = _edge_mlp(src_a, tgt_a, ea_a, w1a, w1b, w1c, b1r, W2, b2r)
    mij_b = _edge_mlp(src_b, tgt_b, ea_b, w1a, w1b, w1c, b1r, W2, b2r)

    row3_a = row_a.reshape(NS, E_SPLIT // NS // 128, 128)
    row3_b = row_b.reshape(NS, ecnt_b // NS // 80, 80)
    agg_a = _scatter_a(mij_a, row3_a)
    agg_b = _scatter_b(mij_b, row3_b)

    h_new = _node_mlp(h, agg_a, agg_b, W3[:D_FEAT], W3[D_FEAT:],
                      b3r, W4, b4r)
    mij = jnp.concatenate([mij_a, mij_b], axis=0)
    return (h_new, mij)


# speedup vs baseline: 3.2381x; 1.0242x over previous
import functools

import jax
import jax.numpy as jnp
from jax import lax
from jax.experimental import pallas as pl
from jax.experimental.pallas import tpu as pltpu
from jax.experimental.pallas import tpu_sc as plsc

N_NODES = 10000
N_EDGES = 320000
D_FEAT = 128
D_EDGE = 16
HIDDEN = 256

NC = 2
NS = 16
NW = NC * NS

GROUPS = ((0, 163840), (163840, 153600), (317440, 2560))
HALF = HIDDEN // NC
NPASS = 2
NHALF = N_NODES // NPASS
ACC_ROWS = NHALF + 8
ZCHUNK = 16
OCHUNK = 40

_sc_mesh = plsc.VectorSubcoreMesh(core_axis_name="c", subcore_axis_name="s")


def _make_gather_ring(e_lo, e_cnt, GCHUNK, RING):
    e_per_w = e_cnt // NW
    giters = e_per_w // GCHUNK
    assert giters % RING == 0

    scratch = [pltpu.VMEM((e_per_w,), jnp.int32),
               pltpu.VMEM((e_per_w,), jnp.int32)]
    scratch += [pltpu.VMEM((GCHUNK, D_FEAT), jnp.float32)] * (2 * RING)
    scratch += [pltpu.SemaphoreType.DMA] * (4 * RING)

    @functools.partial(
        pl.kernel,
        mesh=_sc_mesh,
        out_type=[
            jax.ShapeDtypeStruct((e_cnt, D_FEAT), jnp.float32),
            jax.ShapeDtypeStruct((e_cnt, D_FEAT), jnp.float32),
        ],
        scratch_types=scratch,
    )
    def gather(h_hbm, row_hbm, col_hbm, src_hbm, tgt_hbm, *scr):
        rowi, coli = scr[0], scr[1]
        bufa = scr[2:2 + RING]
        bufb = scr[2 + RING:2 + 2 * RING]
        sga = scr[2 + 2 * RING:2 + 3 * RING]
        sgb = scr[2 + 3 * RING:2 + 4 * RING]
        swa = scr[2 + 4 * RING:2 + 5 * RING]
        swb = scr[2 + 5 * RING:2 + 6 * RING]

        c = lax.axis_index("c")
        s = lax.axis_index("s")
        wid = s * NC + c
        base = wid * e_per_w
        pltpu.sync_copy(row_hbm.at[pl.ds(e_lo + base, e_per_w)], rowi)
        pltpu.sync_copy(col_hbm.at[pl.ds(e_lo + base, e_per_w)], coli)

        def g_issue(j, k):
            off = j * GCHUNK
            pltpu.async_copy(h_hbm.at[rowi.at[pl.ds(off, GCHUNK)]],
                             bufa[k], sga[k])
            pltpu.async_copy(h_hbm.at[coli.at[pl.ds(off, GCHUNK)]],
                             bufb[k], sgb[k])

        def g_wait(k):
            pltpu.make_async_copy(h_hbm.at[pl.ds(0, GCHUNK)], bufa[k],
                                  sga[k]).wait()
            pltpu.make_async_copy(h_hbm.at[pl.ds(0, GCHUNK)], bufb[k],
                                  sgb[k]).wait()

        def w_issue(j, k):
            off = j * GCHUNK
            pltpu.async_copy(bufa[k], src_hbm.at[pl.ds(base + off, GCHUNK)],
                             swa[k])
            pltpu.async_copy(bufb[k], tgt_hbm.at[pl.ds(base + off, GCHUNK)],
                             swb[k])

        def w_wait(k):
            pltpu.make_async_copy(bufa[k], src_hbm.at[pl.ds(0, GCHUNK)],
                                  swa[k]).wait()
            pltpu.make_async_copy(bufb[k], tgt_hbm.at[pl.ds(0, GCHUNK)],
                                  swb[k]).wait()

        g_issue(0, 0)
        g_issue(1, 1)

        def body(t, carry):
            for k in range(RING):
                j = RING * t + k
                g_wait(k)
                w_issue(j, k)
                k2 = (k + 2) % RING
                if k < RING - 2:
                    @pl.when(t > 0)
                    def _():
                        w_wait(k2)

                    g_issue(j + 2, k2)
                else:
                    w_wait(k2)

                    @pl.when(t < giters // RING - 1)
                    def _():
                        g_issue(j + 2, k2)
            return carry

        lax.fori_loop(0, giters // RING, body, 0)
        for jj in range(giters - RING + 2, giters):
            w_wait(jj % RING)

    return gather


def _make_gather(e_lo, e_cnt, GCHUNK):
    e_per_w = e_cnt // NW
    giters = e_per_w // GCHUNK

    @functools.partial(
        pl.kernel,
        mesh=_sc_mesh,
        out_type=[
            jax.ShapeDtypeStruct((e_cnt, D_FEAT), jnp.float32),
            jax.ShapeDtypeStruct((e_cnt, D_FEAT), jnp.float32),
        ],
        scratch_types=[
            pltpu.VMEM((e_per_w,), jnp.int32),
            pltpu.VMEM((e_per_w,), jnp.int32),
            pltpu.VMEM((GCHUNK, D_FEAT), jnp.float32),
            pltpu.VMEM((GCHUNK, D_FEAT), jnp.float32),
            pltpu.VMEM((GCHUNK, D_FEAT), jnp.float32),
            pltpu.VMEM((GCHUNK, D_FEAT), jnp.float32),
            pltpu.SemaphoreType.DMA,
            pltpu.SemaphoreType.DMA,
            pltpu.SemaphoreType.DMA,
            pltpu.SemaphoreType.DMA,
        ],
    )
    def gather(h_hbm, row_hbm, col_hbm, src_hbm, tgt_hbm,
               rowi, coli, bufa0, bufb0, bufa1, bufb1, sa0, sb0, sa1, sb1):
        c = lax.axis_index("c")
        s = lax.axis_index("s")
        wid = s * NC + c
        base = wid * e_per_w
        pltpu.sync_copy(row_hbm.at[pl.ds(e_lo + base, e_per_w)], rowi)
        pltpu.sync_copy(col_hbm.at[pl.ds(e_lo + base, e_per_w)], coli)

        def issue(off, ba, bb, sema, semb):
            pltpu.async_copy(h_hbm.at[rowi.at[pl.ds(off, GCHUNK)]], ba, sema)
            pltpu.async_copy(h_hbm.at[coli.at[pl.ds(off, GCHUNK)]], bb, semb)

        def drain(ba, bb, sema, semb):
            pltpu.make_async_copy(h_hbm.at[pl.ds(0, GCHUNK)], ba, sema).wait()
            pltpu.make_async_copy(h_hbm.at[pl.ds(0, GCHUNK)], bb, semb).wait()

        def write(off, ba, bb):
            pltpu.sync_copy(ba, src_hbm.at[pl.ds(base + off, GCHUNK)])
            pltpu.sync_copy(bb, tgt_hbm.at[pl.ds(base + off, GCHUNK)])

        issue(0, bufa0, bufb0, sa0, sb0)

        def body(t, carry):
            off0 = 2 * t * GCHUNK
            off1 = off0 + GCHUNK
            issue(off1, bufa1, bufb1, sa1, sb1)
            drain(bufa0, bufb0, sa0, sb0)
            write(off0, bufa0, bufb0)

            @pl.when(2 * t + 2 < giters)
            def _():
                issue(off1 + GCHUNK, bufa0, bufb0, sa0, sb0)

            drain(bufa1, bufb1, sa1, sb1)
            write(off1, bufa1, bufb1)
            return carry

        lax.fori_loop(0, giters // 2, body, 0)
        if giters % 2:
            off = (giters - 1) * GCHUNK
            drain(bufa0, bufb0, sa0, sb0)
            write(off, bufa0, bufb0)

    return gather


_gathers = (
    _make_gather_ring(GROUPS[0][0], GROUPS[0][1], 64, 4),
    _make_gather_ring(GROUPS[1][0], GROUPS[1][1], 80, 4),
    _make_gather(GROUPS[2][0], GROUPS[2][1], 80),
)


def _make_scatter_ring(e_cnt, SCHUNK, RING):
    e_per_t = e_cnt // NS
    siters = e_per_t // SCHUNK
    assert siters % RING == 0

    scratch = [pltpu.VMEM((siters, SCHUNK), jnp.int32)]
    scratch += [pltpu.VMEM((1, SCHUNK), jnp.int32)] * RING
    scratch += [pltpu.VMEM((SCHUNK, HALF), jnp.float32)] * RING
    scratch += [pltpu.VMEM((ZCHUNK, HALF), jnp.float32),
                pltpu.VMEM((OCHUNK, HALF), jnp.float32),
                pltpu.VMEM_SHARED((ACC_ROWS, HALF), jnp.float32)]
    scratch += [pltpu.SemaphoreType.DMA] * (2 * RING)

    @functools.partial(
        pl.kernel,
        mesh=_sc_mesh,
        out_type=jax.ShapeDtypeStruct((N_NODES, HIDDEN), jnp.float32),
        scratch_types=scratch,
    )
    def scatter(mij_hbm, row3_hbm, agg_hbm, *scr):
        idxv = scr[0]
        lidx = scr[1:1 + RING]
        bufv = scr[1 + RING:1 + 2 * RING]
        zbuf, obuf, acc = scr[1 + 2 * RING:4 + 2 * RING]
        sr = scr[4 + 2 * RING:4 + 3 * RING]
        ss = scr[4 + 3 * RING:4 + 4 * RING]

        c = lax.axis_index("c")
        s = lax.axis_index("s")

        zero = jnp.zeros((16,), jnp.float32)

        def zrow(i, carry):
            for q in range(HALF // 16):
                zbuf[i, pl.ds(q * 16, 16)] = zero
            return carry

        lax.fori_loop(0, ZCHUNK, zrow, 0)
        pltpu.sync_copy(row3_hbm.at[s], idxv)

        ziters = jnp.where(s == NS - 1, 13, 20)
        oiters = jnp.where(s == NS - 1, 5, 8)
        zstart = s * 320

        for p in range(NPASS):
            nbase = p * NHALF

            def zcp(t, carry):
                pltpu.sync_copy(zbuf, acc.at[pl.ds(zstart + t * ZCHUNK, ZCHUNK)])
                return carry

            lax.fori_loop(0, ziters, zcp, 0)
            plsc.subcore_barrier()

            def rd_issue(j, k):
                ebase = s * e_per_t + j * SCHUNK
                pltpu.async_copy(
                    mij_hbm.at[pl.ds(ebase, SCHUNK), pl.ds(c * HALF, HALF)],
                    bufv[k], sr[k])

            def rd_wait(k):
                pltpu.make_async_copy(
                    mij_hbm.at[pl.ds(0, SCHUNK), pl.ds(0, HALF)], bufv[k],
                    sr[k]).wait()

            trash = NHALF + (lax.iota(jnp.int32, 16) & 7)

            def remap(j, k):
                for q in range(SCHUNK // 16):
                    v = idxv[j, pl.ds(q * 16, 16)] - nbase
                    ok = (v >= 0) & (v < NHALF)
                    lidx[k][0, pl.ds(q * 16, 16)] = jnp.where(ok, v, trash)

            def sc_wait(k):
                pltpu.make_async_copy(bufv[k], acc.at[pl.ds(0, SCHUNK)],
                                      ss[k]).wait()

            rd_issue(0, 0)
            rd_issue(1, 1)

            def ebody(t, carry):
                for k in range(RING):
                    j = RING * t + k
                    rd_wait(k)
                    remap(j, k)
                    pltpu.async_copy(bufv[k], acc.at[lidx[k].at[0]], ss[k],
                                     add=True)
                    k2 = (k + 2) % RING
                    if k < RING - 2:
                        @pl.when(t > 0)
                        def _():
                            sc_wait(k2)

                        rd_issue(j + 2, k2)
                    else:
                        sc_wait(k2)

                        @pl.when(t < siters // RING - 1)
                        def _():
                            rd_issue(j + 2, k2)
                return carry

            lax.fori_loop(0, siters // RING, ebody, 0)
            for jj in range(siters - RING + 2, siters):
                sc_wait(jj % RING)
            plsc.subcore_barrier()

            def obody(t, carry):
                rbase = zstart + t * OCHUNK
                pltpu.sync_copy(acc.at[pl.ds(rbase, OCHUNK)], obuf)
                pltpu.sync_copy(
                    obuf,
                    agg_hbm.at[pl.ds(nbase + rbase, OCHUNK),
                               pl.ds(c * HALF, HALF)])
                return carry

            lax.fori_loop(0, oiters, obody, 0)

    return scatter


def _make_scatter(e_cnt, SCHUNK):
    e_per_t = e_cnt // NS
    siters = e_per_t // SCHUNK

    @functools.partial(
        pl.kernel,
        mesh=_sc_mesh,
        out_type=jax.ShapeDtypeStruct((N_NODES, HIDDEN), jnp.float32),
        scratch_types=[
            pltpu.VMEM((siters, SCHUNK), jnp.int32),
            pltpu.VMEM((1, SCHUNK), jnp.int32),
            pltpu.VMEM((1, SCHUNK), jnp.int32),
            pltpu.VMEM((SCHUNK, HALF), jnp.float32),
            pltpu.VMEM((SCHUNK, HALF), jnp.float32),
            pltpu.VMEM((ZCHUNK, HALF), jnp.float32),
            pltpu.VMEM((OCHUNK, HALF), jnp.float32),
            pltpu.VMEM_SHARED((ACC_ROWS, HALF), jnp.float32),
            pltpu.SemaphoreType.DMA,
            pltpu.SemaphoreType.DMA,
            pltpu.SemaphoreType.DMA,
            pltpu.SemaphoreType.DMA,
        ],
    )
    def scatter(mij_hbm, row3_hbm, agg_hbm, idxv, lidx0, lidx1,
                bufv0, bufv1, zbuf, obuf, acc, sr0, sr1, ss0, ss1):
        c = lax.axis_index("c")
        s = lax.axis_index("s")

        zero = jnp.zeros((16,), jnp.float32)

        def zrow(i, carry):
            for q in range(HALF // 16):
                zbuf[i, pl.ds(q * 16, 16)] = zero
            return carry

        lax.fori_loop(0, ZCHUNK, zrow, 0)

        pltpu.sync_copy(row3_hbm.at[s], idxv)

        ziters = jnp.where(s == NS - 1, 13, 20)
        oiters = jnp.where(s == NS - 1, 5, 8)
        zstart = s * 320

        for p in range(NPASS):
            nbase = p * NHALF

            def zcp(t, carry):
                pltpu.sync_copy(zbuf, acc.at[pl.ds(zstart + t * ZCHUNK, ZCHUNK)])
                return carry

            lax.fori_loop(0, ziters, zcp, 0)
            plsc.subcore_barrier()

            def rd_issue(j, buf, sem):
                ebase = s * e_per_t + j * SCHUNK
                pltpu.async_copy(
                    mij_hbm.at[pl.ds(ebase, SCHUNK), pl.ds(c * HALF, HALF)],
                    buf, sem)

            def rd_wait(buf, sem):
                pltpu.make_async_copy(
                    mij_hbm.at[pl.ds(0, SCHUNK), pl.ds(0, HALF)], buf,
                    sem).wait()

            trash = NHALF + (lax.iota(jnp.int32, 16) & 7)

            def remap(j, lidx):
                for k in range(SCHUNK // 16):
                    v = idxv[j, pl.ds(k * 16, 16)] - nbase
                    ok = (v >= 0) & (v < NHALF)
                    lidx[0, pl.ds(k * 16, 16)] = jnp.where(ok, v, trash)

            def sc_wait(buf, sem):
                pltpu.make_async_copy(buf, acc.at[pl.ds(0, SCHUNK)], sem).wait()

            rd_issue(0, bufv0, sr0)

            def ebody(t, carry):
                j0 = 2 * t
                j1 = j0 + 1

                @pl.when(t > 0)
                def _():
                    sc_wait(bufv1, ss1)

                rd_issue(j1, bufv1, sr1)
                rd_wait(bufv0, sr0)
                remap(j0, lidx0)
                pltpu.async_copy(bufv0, acc.at[lidx0.at[0]], ss0, add=True)
                rd_wait(bufv1, sr1)
                remap(j1, lidx1)
                pltpu.async_copy(bufv1, acc.at[lidx1.at[0]], ss1, add=True)
                sc_wait(bufv0, ss0)

                @pl.when(t < siters // 2 - 1)
                def _():
                    rd_issue(j0 + 2, bufv0, sr0)

                return carry

            lax.fori_loop(0, siters // 2, ebody, 0)
            sc_wait(bufv1, ss1)
            plsc.subcore_barrier()

            def obody(t, carry):
                rbase = zstart + t * OCHUNK
                pltpu.sync_copy(acc.at[pl.ds(rbase, OCHUNK)], obuf)
                pltpu.sync_copy(
                    obuf,
                    agg_hbm.at[pl.ds(nbase + rbase, OCHUNK),
                               pl.ds(c * HALF, HALF)])
                return carry

            lax.fori_loop(0, oiters, obody, 0)

    return scatter


_scatters = (
    _make_scatter_ring(GROUPS[0][1], 128, 4),
    _make_scatter_ring(GROUPS[1][1], 96, 4),
    _make_scatter(GROUPS[2][1], 80),
)
_SCHUNKS = (128, 96, 80)


BE = 2560


def _edge_mlp_body(src_ref, tgt_ref, ea_ref, w1a, w1b, w1c, b1r, w2r, b2r,
                   out_ref):
    x = (jnp.dot(src_ref[...], w1a[...], preferred_element_type=jnp.float32)
         + jnp.dot(tgt_ref[...], w1b[...], preferred_element_type=jnp.float32)
         + jnp.dot(ea_ref[...], w1c[...], preferred_element_type=jnp.float32)
         + b1r[...])
    x = x * jax.nn.sigmoid(x)
    y = jnp.dot(x, w2r[...], preferred_element_type=jnp.float32) + b2r[...]
    out_ref[...] = y * jax.nn.sigmoid(y)


def _edge_mlp(src, tgt, ea, blk_off, w1a, w1b, w1c, b1, w2, b2):
    e_cnt = src.shape[0]
    grid = (e_cnt // BE,)
    full = lambda shape: pl.BlockSpec(shape, lambda i: (0, 0))
    return pl.pallas_call(
        _edge_mlp_body,
        grid=grid,
        in_specs=[
            pl.BlockSpec((BE, D_FEAT), lambda i: (i, 0)),
            pl.BlockSpec((BE, D_FEAT), lambda i: (i, 0)),
            pl.BlockSpec((BE, D_EDGE), lambda i: (i + blk_off, 0)),
            full((D_FEAT, HIDDEN)),
            full((D_FEAT, HIDDEN)),
            full((D_EDGE, HIDDEN)),
            full((1, HIDDEN)),
            full((HIDDEN, HIDDEN)),
            full((1, HIDDEN)),
        ],
        out_specs=pl.BlockSpec((BE, HIDDEN), lambda i: (i, 0)),
        out_shape=jax.ShapeDtypeStruct((e_cnt, HIDDEN), jnp.float32),
    )(src, tgt, ea, w1a, w1b, w1c, b1, w2, b2)


BN = 2000


def _node_mlp_body(h_ref, agga_ref, aggb_ref, aggc_ref, w3a, w3b, b3r, w4r,
                   b4r, out_ref):
    agg = agga_ref[...] + aggb_ref[...] + aggc_ref[...]
    x = (jnp.dot(h_ref[...], w3a[...], preferred_element_type=jnp.float32)
         + jnp.dot(agg, w3b[...], preferred_element_type=jnp.float32)
         + b3r[...])
    x = x * jax.nn.sigmoid(x)
    out_ref[...] = (h_ref[...]
                    + jnp.dot(x, w4r[...], preferred_element_type=jnp.float32)
                    + b4r[...])


def _node_mlp(h, agga, aggb, aggc, w3a, w3b, b3, w4, b4):
    grid = (N_NODES // BN,)
    full = lambda shape: pl.BlockSpec(shape, lambda i: (0, 0))
    return pl.pallas_call(
        _node_mlp_body,
        grid=grid,
        in_specs=[
            pl.BlockSpec((BN, D_FEAT), lambda i: (i, 0)),
            pl.BlockSpec((BN, HIDDEN), lambda i: (i, 0)),
            pl.BlockSpec((BN, HIDDEN), lambda i: (i, 0)),
            pl.BlockSpec((BN, HIDDEN), lambda i: (i, 0)),
            full((D_FEAT, HIDDEN)),
            full((HIDDEN, HIDDEN)),
            full((1, HIDDEN)),
            full((HIDDEN, D_FEAT)),
            full((1, D_FEAT)),
        ],
        out_specs=pl.BlockSpec((BN, D_FEAT), lambda i: (i, 0)),
        out_shape=jax.ShapeDtypeStruct((N_NODES, D_FEAT), jnp.float32),
    )(h, agga, aggb, aggc, w3a, w3b, b3, w4, b4)


def kernel(h, edge_index, edge_attr, W1, b1, W2, b2, W3, b3, W4, b4):
    row = edge_index[0].astype(jnp.int32)
    col = edge_index[1].astype(jnp.int32)

    w1a, w1b, w1c = W1[:D_FEAT], W1[D_FEAT:2 * D_FEAT], W1[2 * D_FEAT:]
    b1r, b2r, b3r, b4r = (b.reshape(1, -1) for b in (b1, b2, b3, b4))

    mijs, aggs = [], []
    for g, (e_lo, e_cnt) in enumerate(GROUPS):
        src, tgt = _gathers[g](h, row, col)
        mij = _edge_mlp(src, tgt, edge_attr, e_lo // BE,
                        w1a, w1b, w1c, b1r, W2, b2r)
        sch = _SCHUNKS[g]
        row3 = lax.slice(row, (e_lo,), (e_lo + e_cnt,)).reshape(
            NS, e_cnt // NS // sch, sch)
        aggs.append(_scatters[g](mij, row3))
        mijs.append(mij)

    h_new = _node_mlp(h, aggs[0], aggs[1], aggs[2], W3[:D_FEAT], W3[D_FEAT:],
                      b3r, W4, b4r)
    mij = jnp.concatenate(mijs, axis=0)
    return (h_new, mij)


# speedup vs baseline: 3.2395x; 1.0004x over previous
import functools

import jax
import jax.numpy as jnp
from jax import lax
from jax.experimental import pallas as pl
from jax.experimental.pallas import tpu as pltpu
from jax.experimental.pallas import tpu_sc as plsc

N_NODES = 10000
N_EDGES = 320000
D_FEAT = 128
D_EDGE = 16
HIDDEN = 256

NC = 2
NS = 16
NW = NC * NS

GROUPS = ((0, 163840), (163840, 153600), (317440, 2560))
HALF = HIDDEN // NC
NPASS = 2
NHALF = N_NODES // NPASS
ACC_ROWS = NHALF + 8
ZCHUNK = 16
OCHUNK = 40

_sc_mesh = plsc.VectorSubcoreMesh(core_axis_name="c", subcore_axis_name="s")


def _make_gather_ring(e_lo, e_cnt, GCHUNK, RING):
    e_per_w = e_cnt // NW
    giters = e_per_w // GCHUNK
    assert giters % RING == 0

    scratch = [pltpu.VMEM((e_per_w,), jnp.int32),
               pltpu.VMEM((e_per_w,), jnp.int32)]
    scratch += [pltpu.VMEM((GCHUNK, D_FEAT), jnp.float32)] * (2 * RING)
    scratch += [pltpu.SemaphoreType.DMA] * (4 * RING)

    @functools.partial(
        pl.kernel,
        mesh=_sc_mesh,
        out_type=[
            jax.ShapeDtypeStruct((e_cnt, D_FEAT), jnp.float32),
            jax.ShapeDtypeStruct((e_cnt, D_FEAT), jnp.float32),
        ],
        scratch_types=scratch,
    )
    def gather(h_hbm, row_hbm, col_hbm, src_hbm, tgt_hbm, *scr):
        rowi, coli = scr[0], scr[1]
        bufa = scr[2:2 + RING]
        bufb = scr[2 + RING:2 + 2 * RING]
        sga = scr[2 + 2 * RING:2 + 3 * RING]
        sgb = scr[2 + 3 * RING:2 + 4 * RING]
        swa = scr[2 + 4 * RING:2 + 5 * RING]
        swb = scr[2 + 5 * RING:2 + 6 * RING]

        c = lax.axis_index("c")
        s = lax.axis_index("s")
        wid = s * NC + c
        base = wid * e_per_w
        pltpu.sync_copy(row_hbm.at[pl.ds(e_lo + base, e_per_w)], rowi)
        pltpu.sync_copy(col_hbm.at[pl.ds(e_lo + base, e_per_w)], coli)

        def g_issue(j, k):
            off = j * GCHUNK
            pltpu.async_copy(h_hbm.at[rowi.at[pl.ds(off, GCHUNK)]],
                             bufa[k], sga[k])
            pltpu.async_copy(h_hbm.at[coli.at[pl.ds(off, GCHUNK)]],
                             bufb[k], sgb[k])

        def g_wait(k):
            pltpu.make_async_copy(h_hbm.at[pl.ds(0, GCHUNK)], bufa[k],
                                  sga[k]).wait()
            pltpu.make_async_copy(h_hbm.at[pl.ds(0, GCHUNK)], bufb[k],
                                  sgb[k]).wait()

        def w_issue(j, k):
            off = j * GCHUNK
            pltpu.async_copy(bufa[k], src_hbm.at[pl.ds(base + off, GCHUNK)],
                             swa[k])
            pltpu.async_copy(bufb[k], tgt_hbm.at[pl.ds(base + off, GCHUNK)],
                             swb[k])

        def w_wait(k):
            pltpu.make_async_copy(bufa[k], src_hbm.at[pl.ds(0, GCHUNK)],
                                  swa[k]).wait()
            pltpu.make_async_copy(bufb[k], tgt_hbm.at[pl.ds(0, GCHUNK)],
                                  swb[k]).wait()

        g_issue(0, 0)
        g_issue(1, 1)

        def body(t, carry):
            for k in range(RING):
                j = RING * t + k
                g_wait(k)
                w_issue(j, k)
                k2 = (k + 2) % RING
                if k < RING - 2:
                    @pl.when(t > 0)
                    def _():
                        w_wait(k2)

                    g_issue(j + 2, k2)
                else:
                    w_wait(k2)

                    @pl.when(t < giters // RING - 1)
                    def _():
                        g_issue(j + 2, k2)
            return carry

        lax.fori_loop(0, giters // RING, body, 0)
        for jj in range(giters - RING + 2, giters):
            w_wait(jj % RING)

    return gather


def _make_gather(e_lo, e_cnt, GCHUNK):
    e_per_w = e_cnt // NW
    giters = e_per_w // GCHUNK

    @functools.partial(
        pl.kernel,
        mesh=_sc_mesh,
        out_type=[
            jax.ShapeDtypeStruct((e_cnt, D_FEAT), jnp.float32),
            jax.ShapeDtypeStruct((e_cnt, D_FEAT), jnp.float32),
        ],
        scratch_types=[
            pltpu.VMEM((e_per_w,), jnp.int32),
            pltpu.VMEM((e_per_w,), jnp.int32),
            pltpu.VMEM((GCHUNK, D_FEAT), jnp.float32),
            pltpu.VMEM((GCHUNK, D_FEAT), jnp.float32),
            pltpu.VMEM((GCHUNK, D_FEAT), jnp.float32),
            pltpu.VMEM((GCHUNK, D_FEAT), jnp.float32),
            pltpu.SemaphoreType.DMA,
            pltpu.SemaphoreType.DMA,
            pltpu.SemaphoreType.DMA,
            pltpu.SemaphoreType.DMA,
        ],
    )
    def gather(h_hbm, row_hbm, col_hbm, src_hbm, tgt_hbm,
               rowi, coli, bufa0, bufb0, bufa1, bufb1, sa0, sb0, sa1, sb1):
        c = lax.axis_index("c")
        s = lax.axis_index("s")
        wid = s * NC + c
        base = wid * e_per_w
        pltpu.sync_copy(row_hbm.at[pl.ds(e_lo + base, e_per_w)], rowi)
        pltpu.sync_copy(col_hbm.at[pl.ds(e_lo + base, e_per_w)], coli)

        def issue(off, ba, bb, sema, semb):
            pltpu.async_copy(h_hbm.at[rowi.at[pl.ds(off, GCHUNK)]], ba, sema)
            pltpu.async_copy(h_hbm.at[coli.at[pl.ds(off, GCHUNK)]], bb, semb)

        def drain(ba, bb, sema, semb):
            pltpu.make_async_copy(h_hbm.at[pl.ds(0, GCHUNK)], ba, sema).wait()
            pltpu.make_async_copy(h_hbm.at[pl.ds(0, GCHUNK)], bb, semb).wait()

        def write(off, ba, bb):
            pltpu.sync_copy(ba, src_hbm.at[pl.ds(base + off, GCHUNK)])
            pltpu.sync_copy(bb, tgt_hbm.at[pl.ds(base + off, GCHUNK)])

        issue(0, bufa0, bufb0, sa0, sb0)

        def body(t, carry):
            off0 = 2 * t * GCHUNK
            off1 = off0 + GCHUNK
            issue(off1, bufa1, bufb1, sa1, sb1)
            drain(bufa0, bufb0, sa0, sb0)
            write(off0, bufa0, bufb0)

            @pl.when(2 * t + 2 < giters)
            def _():
                issue(off1 + GCHUNK, bufa0, bufb0, sa0, sb0)

            drain(bufa1, bufb1, sa1, sb1)
            write(off1, bufa1, bufb1)
            return carry

        lax.fori_loop(0, giters // 2, body, 0)
        if giters % 2:
            off = (giters - 1) * GCHUNK
            drain(bufa0, bufb0, sa0, sb0)
            write(off, bufa0, bufb0)

    return gather


_gathers = (
    _make_gather_ring(GROUPS[0][0], GROUPS[0][1], 64, 4),
    _make_gather(GROUPS[1][0], GROUPS[1][1], 80),
    _make_gather(GROUPS[2][0], GROUPS[2][1], 80),
)


def _make_scatter_ring(e_cnt, SCHUNK, RING):
    e_per_t = e_cnt // NS
    siters = e_per_t // SCHUNK
    assert siters % RING == 0

    scratch = [pltpu.VMEM((siters, SCHUNK), jnp.int32)]
    scratch += [pltpu.VMEM((1, SCHUNK), jnp.int32)] * RING
    scratch += [pltpu.VMEM((SCHUNK, HALF), jnp.float32)] * RING
    scratch += [pltpu.VMEM((ZCHUNK, HALF), jnp.float32),
                pltpu.VMEM((OCHUNK, HALF), jnp.float32),
                pltpu.VMEM_SHARED((ACC_ROWS, HALF), jnp.float32)]
    scratch += [pltpu.SemaphoreType.DMA] * (2 * RING)

    @functools.partial(
        pl.kernel,
        mesh=_sc_mesh,
        out_type=jax.ShapeDtypeStruct((N_NODES, HIDDEN), jnp.float32),
        scratch_types=scratch,
    )
    def scatter(mij_hbm, row3_hbm, agg_hbm, *scr):
        idxv = scr[0]
        lidx = scr[1:1 + RING]
        bufv = scr[1 + RING:1 + 2 * RING]
        zbuf, obuf, acc = scr[1 + 2 * RING:4 + 2 * RING]
        sr = scr[4 + 2 * RING:4 + 3 * RING]
        ss = scr[4 + 3 * RING:4 + 4 * RING]

        c = lax.axis_index("c")
        s = lax.axis_index("s")

        zero = jnp.zeros((16,), jnp.float32)

        def zrow(i, carry):
            for q in range(HALF // 16):
                zbuf[i, pl.ds(q * 16, 16)] = zero
            return carry

        lax.fori_loop(0, ZCHUNK, zrow, 0)
        pltpu.sync_copy(row3_hbm.at[s], idxv)

        ziters = jnp.where(s == NS - 1, 13, 20)
        oiters = jnp.where(s == NS - 1, 5, 8)
        zstart = s * 320

        for p in range(NPASS):
            nbase = p * NHALF

            def zcp(t, carry):
                pltpu.sync_copy(zbuf, acc.at[pl.ds(zstart + t * ZCHUNK, ZCHUNK)])
                return carry

            lax.fori_loop(0, ziters, zcp, 0)
            plsc.subcore_barrier()

            def rd_issue(j, k):
                ebase = s * e_per_t + j * SCHUNK
                pltpu.async_copy(
                    mij_hbm.at[pl.ds(ebase, SCHUNK), pl.ds(c * HALF, HALF)],
                    bufv[k], sr[k])

            def rd_wait(k):
                pltpu.make_async_copy(
                    mij_hbm.at[pl.ds(0, SCHUNK), pl.ds(0, HALF)], bufv[k],
                    sr[k]).wait()

            trash = NHALF + (lax.iota(jnp.int32, 16) & 7)

            def remap(j, k):
                for q in range(SCHUNK // 16):
                    v = idxv[j, pl.ds(q * 16, 16)] - nbase
                    ok = (v >= 0) & (v < NHALF)
                    lidx[k][0, pl.ds(q * 16, 16)] = jnp.where(ok, v, trash)

            def sc_wait(k):
                pltpu.make_async_copy(bufv[k], acc.at[pl.ds(0, SCHUNK)],
                                      ss[k]).wait()

            rd_issue(0, 0)
            rd_issue(1, 1)

            def ebody(t, carry):
                for k in range(RING):
                    j = RING * t + k
                    rd_wait(k)
                    remap(j, k)
                    pltpu.async_copy(bufv[k], acc.at[lidx[k].at[0]], ss[k],
                                     add=True)
                    k2 = (k + 2) % RING
                    if k < RING - 2:
                        @pl.when(t > 0)
                        def _():
                            sc_wait(k2)

                        rd_issue(j + 2, k2)
                    else:
                        sc_wait(k2)

                        @pl.when(t < siters // RING - 1)
                        def _():
                            rd_issue(j + 2, k2)
                return carry

            lax.fori_loop(0, siters // RING, ebody, 0)
            for jj in range(siters - RING + 2, siters):
                sc_wait(jj % RING)
            plsc.subcore_barrier()

            def obody(t, carry):
                rbase = zstart + t * OCHUNK
                pltpu.sync_copy(acc.at[pl.ds(rbase, OCHUNK)], obuf)
                pltpu.sync_copy(
                    obuf,
                    agg_hbm.at[pl.ds(nbase + rbase, OCHUNK),
                               pl.ds(c * HALF, HALF)])
                return carry

            lax.fori_loop(0, oiters, obody, 0)

    return scatter


def _make_scatter(e_cnt, SCHUNK):
    e_per_t = e_cnt // NS
    siters = e_per_t // SCHUNK

    @functools.partial(
        pl.kernel,
        mesh=_sc_mesh,
        out_type=jax.ShapeDtypeStruct((N_NODES, HIDDEN), jnp.float32),
        scratch_types=[
            pltpu.VMEM((siters, SCHUNK), jnp.int32),
            pltpu.VMEM((1, SCHUNK), jnp.int32),
            pltpu.VMEM((1, SCHUNK), jnp.int32),
            pltpu.VMEM((SCHUNK, HALF), jnp.float32),
            pltpu.VMEM((SCHUNK, HALF), jnp.float32),
            pltpu.VMEM((ZCHUNK, HALF), jnp.float32),
            pltpu.VMEM((OCHUNK, HALF), jnp.float32),
            pltpu.VMEM_SHARED((ACC_ROWS, HALF), jnp.float32),
            pltpu.SemaphoreType.DMA,
            pltpu.SemaphoreType.DMA,
            pltpu.SemaphoreType.DMA,
            pltpu.SemaphoreType.DMA,
        ],
    )
    def scatter(mij_hbm, row3_hbm, agg_hbm, idxv, lidx0, lidx1,
                bufv0, bufv1, zbuf, obuf, acc, sr0, sr1, ss0, ss1):
        c = lax.axis_index("c")
        s = lax.axis_index("s")

        zero = jnp.zeros((16,), jnp.float32)

        def zrow(i, carry):
            for q in range(HALF // 16):
                zbuf[i, pl.ds(q * 16, 16)] = zero
            return carry

        lax.fori_loop(0, ZCHUNK, zrow, 0)

        pltpu.sync_copy(row3_hbm.at[s], idxv)

        ziters = jnp.where(s == NS - 1, 13, 20)
        oiters = jnp.where(s == NS - 1, 5, 8)
        zstart = s * 320

        for p in range(NPASS):
            nbase = p * NHALF

            def zcp(t, carry):
                pltpu.sync_copy(zbuf, acc.at[pl.ds(zstart + t * ZCHUNK, ZCHUNK)])
                return carry

            lax.fori_loop(0, ziters, zcp, 0)
            plsc.subcore_barrier()

            def rd_issue(j, buf, sem):
                ebase = s * e_per_t + j * SCHUNK
                pltpu.async_copy(
                    mij_hbm.at[pl.ds(ebase, SCHUNK), pl.ds(c * HALF, HALF)],
                    buf, sem)

            def rd_wait(buf, sem):
                pltpu.make_async_copy(
                    mij_hbm.at[pl.ds(0, SCHUNK), pl.ds(0, HALF)], buf,
                    sem).wait()

            trash = NHALF + (lax.iota(jnp.int32, 16) & 7)

            def remap(j, lidx):
                for k in range(SCHUNK // 16):
                    v = idxv[j, pl.ds(k * 16, 16)] - nbase
                    ok = (v >= 0) & (v < NHALF)
                    lidx[0, pl.ds(k * 16, 16)] = jnp.where(ok, v, trash)

            def sc_wait(buf, sem):
                pltpu.make_async_copy(buf, acc.at[pl.ds(0, SCHUNK)], sem).wait()

            rd_issue(0, bufv0, sr0)

            def ebody(t, carry):
                j0 = 2 * t
                j1 = j0 + 1

                @pl.when(t > 0)
                def _():
                    sc_wait(bufv1, ss1)

                rd_issue(j1, bufv1, sr1)
                rd_wait(bufv0, sr0)
                remap(j0, lidx0)
                pltpu.async_copy(bufv0, acc.at[lidx0.at[0]], ss0, add=True)
                rd_wait(bufv1, sr1)
                remap(j1, lidx1)
                pltpu.async_copy(bufv1, acc.at[lidx1.at[0]], ss1, add=True)
                sc_wait(bufv0, ss0)

                @pl.when(t < siters // 2 - 1)
                def _():
                    rd_issue(j0 + 2, bufv0, sr0)

                return carry

            lax.fori_loop(0, siters // 2, ebody, 0)
            sc_wait(bufv1, ss1)
            plsc.subcore_barrier()

            def obody(t, carry):
                rbase = zstart + t * OCHUNK
                pltpu.sync_copy(acc.at[pl.ds(rbase, OCHUNK)], obuf)
                pltpu.sync_copy(
                    obuf,
                    agg_hbm.at[pl.ds(nbase + rbase, OCHUNK),
                               pl.ds(c * HALF, HALF)])
                return carry

            lax.fori_loop(0, oiters, obody, 0)

    return scatter


_scatters = (
    _make_scatter_ring(GROUPS[0][1], 128, 4),
    _make_scatter_ring(GROUPS[1][1], 96, 4),
    _make_scatter(GROUPS[2][1], 80),
)
_SCHUNKS = (128, 96, 80)


BE = 2560


def _edge_mlp_body(src_ref, tgt_ref, ea_ref, w1a, w1b, w1c, b1r, w2r, b2r,
                   out_ref):
    x = (jnp.dot(src_ref[...], w1a[...], preferred_element_type=jnp.float32)
         + jnp.dot(tgt_ref[...], w1b[...], preferred_element_type=jnp.float32)
         + jnp.dot(ea_ref[...], w1c[...], preferred_element_type=jnp.float32)
         + b1r[...])
    x = x * jax.nn.sigmoid(x)
    y = jnp.dot(x, w2r[...], preferred_element_type=jnp.float32) + b2r[...]
    out_ref[...] = y * jax.nn.sigmoid(y)


def _edge_mlp(src, tgt, ea, blk_off, w1a, w1b, w1c, b1, w2, b2):
    e_cnt = src.shape[0]
    grid = (e_cnt // BE,)
    full = lambda shape: pl.BlockSpec(shape, lambda i: (0, 0))
    return pl.pallas_call(
        _edge_mlp_body,
        grid=grid,
        in_specs=[
            pl.BlockSpec((BE, D_FEAT), lambda i: (i, 0)),
            pl.BlockSpec((BE, D_FEAT), lambda i: (i, 0)),
            pl.BlockSpec((BE, D_EDGE), lambda i: (i + blk_off, 0)),
            full((D_FEAT, HIDDEN)),
            full((D_FEAT, HIDDEN)),
            full((D_EDGE, HIDDEN)),
            full((1, HIDDEN)),
            full((HIDDEN, HIDDEN)),
            full((1, HIDDEN)),
        ],
        out_specs=pl.BlockSpec((BE, HIDDEN), lambda i: (i, 0)),
        out_shape=jax.ShapeDtypeStruct((e_cnt, HIDDEN), jnp.float32),
    )(src, tgt, ea, w1a, w1b, w1c, b1, w2, b2)


BN = 2000


def _node_mlp_body(h_ref, agga_ref, aggb_ref, aggc_ref, w3a, w3b, b3r, w4r,
                   b4r, out_ref):
    agg = agga_ref[...] + aggb_ref[...] + aggc_ref[...]
    x = (jnp.dot(h_ref[...], w3a[...], preferred_element_type=jnp.float32)
         + jnp.dot(agg, w3b[...], preferred_element_type=jnp.float32)
         + b3r[...])
    x = x * jax.nn.sigmoid(x)
    out_ref[...] = (h_ref[...]
                    + jnp.dot(x, w4r[...], preferred_element_type=jnp.float32)
                    + b4r[...])


def _node_mlp(h, agga, aggb, aggc, w3a, w3b, b3, w4, b4):
    grid = (N_NODES // BN,)
    full = lambda shape: pl.BlockSpec(shape, lambda i: (0, 0))
    return pl.pallas_call(
        _node_mlp_body,
        grid=grid,
        in_specs=[
            pl.BlockSpec((BN, D_FEAT), lambda i: (i, 0)),
            pl.BlockSpec((BN, HIDDEN), lambda i: (i, 0)),
            pl.BlockSpec((BN, HIDDEN), lambda i: (i, 0)),
            pl.BlockSpec((BN, HIDDEN), lambda i: (i, 0)),
            full((D_FEAT, HIDDEN)),
            full((HIDDEN, HIDDEN)),
            full((1, HIDDEN)),
            full((HIDDEN, D_FEAT)),
            full((1, D_FEAT)),
        ],
        out_specs=pl.BlockSpec((BN, D_FEAT), lambda i: (i, 0)),
        out_shape=jax.ShapeDtypeStruct((N_NODES, D_FEAT), jnp.float32),
    )(h, agga, aggb, aggc, w3a, w3b, b3, w4, b4)


def kernel(h, edge_index, edge_attr, W1, b1, W2, b2, W3, b3, W4, b4):
    row = edge_index[0].astype(jnp.int32)
    col = edge_index[1].astype(jnp.int32)

    w1a, w1b, w1c = W1[:D_FEAT], W1[D_FEAT:2 * D_FEAT], W1[2 * D_FEAT:]
    b1r, b2r, b3r, b4r = (b.reshape(1, -1) for b in (b1, b2, b3, b4))

    mijs, aggs = [], []
    for g, (e_lo, e_cnt) in enumerate(GROUPS):
        src, tgt = _gathers[g](h, row, col)
        mij = _edge_mlp(src, tgt, edge_attr, e_lo // BE,
                        w1a, w1b, w1c, b1r, W2, b2r)
        sch = _SCHUNKS[g]
        row3 = lax.slice(row, (e_lo,), (e_lo + e_cnt,)).reshape(
            NS, e_cnt // NS // sch, sch)
        aggs.append(_scatters[g](mij, row3))
        mijs.append(mij)

    h_new = _node_mlp(h, aggs[0], aggs[1], aggs[2], W3[:D_FEAT], W3[D_FEAT:],
                      b3r, W4, b4r)
    mij = jnp.concatenate(mijs, axis=0)
    return (h_new, mij)
